# baseline (speedup 1.0000x reference)
import functools

import jax
import jax.numpy as jnp
from jax import lax
from jax.experimental import pallas as pl
from jax.experimental.pallas import tpu as pltpu
from jax.experimental.pallas import tpu_sc as plsc

Bb, Nn, DFf, KK = 4, 2048, 512, 16
EPS = 1e-5
F32 = jnp.float32


def _hi(x):
    return x


def _dotT(a, b):
    return lax.dot_general(a, b, (((1,), (1,)), ((), ())),
                           preferred_element_type=F32)


def _dot(a, b):
    return lax.dot_general(a, b, (((1,), (0,)), ((), ())),
                           preferred_element_type=F32)


_RBK = 256


def _knn_body(ptsb_ref, ptsf_ref, ones_ref, idx_ref, idx2_ref):
    pb = ptsb_ref[0]
    pf = ptsf_ref[0]
    sq_b = jnp.sum(pb * pb, axis=1, keepdims=True)
    sq_f = _dotT(ones_ref[...], pf * pf)
    g = _dotT(pb, pf)
    d = sq_b + sq_f - 2.0 * g
    col = lax.broadcasted_iota(jnp.int32, (_RBK, Nn), 1)
    big = jnp.float32(1e30)
    i = pl.program_id(1)
    n_self = i * _RBK + lax.broadcasted_iota(jnp.int32, (_RBK, 1), 0)
    dd = jnp.where(col == n_self, big, d)
    ams = []
    for _ in range(KK):
        dmin = jnp.min(dd, axis=1, keepdims=True)
        eqm = dd == dmin
        am = jnp.min(jnp.where(eqm, col, Nn), axis=1, keepdims=True)
        ams.append(am)
        dd = jnp.where(eqm, big, dd)
    v = jnp.concatenate(ams, axis=1)
    idx_ref[0] = v
    idx2_ref[0] = jnp.concatenate([n_self, v[:, :KK - 1]], axis=1)


def _knn(pts8):
    grid = (Bb, Nn // _RBK)
    out = pl.pallas_call(
        _knn_body,
        grid=grid,
        in_specs=[
            pl.BlockSpec((1, _RBK, 8), lambda b, i: (b, i, 0)),
            pl.BlockSpec((1, Nn, 8), lambda b, i: (b, 0, 0)),
            pl.BlockSpec((1, 8), lambda b, i: (0, 0)),
        ],
        out_specs=[
            pl.BlockSpec((1, _RBK, KK), lambda b, i: (b, i, 0)),
            pl.BlockSpec((1, _RBK, KK), lambda b, i: (b, i, 0)),
        ],
        out_shape=[
            jax.ShapeDtypeStruct((Bb, Nn, KK), jnp.int32),
            jax.ShapeDtypeStruct((Bb, Nn, KK), jnp.int32),
        ],
    )(pts8, pts8, jnp.ones((1, 8), F32))
    return out


def _mlp1_body(x_ref, w1_ref, b1_ref, w2_ref, b2_ref, o_ref, oh_ref, ol_ref):
    h = jnp.maximum(_dot(x_ref[...], w1_ref[...]) + b1_ref[...], 0.0)
    y = _dot(h, w2_ref[...]) + b2_ref[...]
    o_ref[...] = y
    yh = y.astype(jnp.bfloat16)
    oh_ref[...] = yh
    ol_ref[...] = (y - yh.astype(F32)).astype(jnp.bfloat16)


def _mlp1(xrows, w1t, b1, w2t, b2):
    R = xrows.shape[0]
    RB = 1024
    return pl.pallas_call(
        _mlp1_body,
        grid=(R // RB,),
        in_specs=[
            pl.BlockSpec((RB, 8), lambda i: (i, 0)),
            pl.BlockSpec(w1t.shape, lambda i: (0, 0)),
            pl.BlockSpec(b1.shape, lambda i: (0, 0)),
            pl.BlockSpec(w2t.shape, lambda i: (0, 0)),
            pl.BlockSpec(b2.shape, lambda i: (0, 0)),
        ],
        out_specs=[
            pl.BlockSpec((RB, 128), lambda i: (i, 0)),
            pl.BlockSpec((RB, 128), lambda i: (i, 0)),
            pl.BlockSpec((RB, 128), lambda i: (i, 0)),
        ],
        out_shape=[
            jax.ShapeDtypeStruct((R, 128), F32),
            jax.ShapeDtypeStruct((R, 128), jnp.bfloat16),
            jax.ShapeDtypeStruct((R, 128), jnp.bfloat16),
        ],
    )(xrows, w1t, b1, w2t, b2)


_RBS = 256


def _sage_body(idx_ref, xh_ref, xl_ref, xb_ref, wl_ref, wr_ref, cb_ref,
               sc_ref, sh_ref, o_ref, oh_ref, ol_ref, *, residual,
               emit_split):
    idx = idx_ref[0]
    xb = xb_ref[0]
    col = lax.broadcasted_iota(jnp.int32, (_RBS, Nn), 1)
    m = col == idx[:, 0:1]
    for k in range(1, KK):
        m = m | (col == idx[:, k:k + 1])
    acc = jnp.where(m, 1.0, 0.0).astype(jnp.bfloat16)
    mean = (_dot(acc, xh_ref[0]) + _dot(acc, xl_ref[0])) * (1.0 / KK)
    y = _dot(mean, wl_ref[...]) + cb_ref[...] + _dot(xb, wr_ref[...])
    y = jnp.maximum(y * sc_ref[...] + sh_ref[...], 0.0)
    if residual:
        y = y + xb
    o_ref[0] = y
    if emit_split:
        yh = y.astype(jnp.bfloat16)
        oh_ref[0] = yh
        ol_ref[0] = (y - yh.astype(F32)).astype(jnp.bfloat16)
    else:
        oh_ref[0] = jnp.zeros((_RBS, 128), jnp.bfloat16)
        ol_ref[0] = jnp.zeros((_RBS, 128), jnp.bfloat16)


def _sage(xh, xl, xb, idx, wlt, bl, wrt, bn_s, bn_t, residual, emit_split):
    body = functools.partial(_sage_body, residual=residual,
                             emit_split=emit_split)
    return pl.pallas_call(
        body,
        grid=(Bb, Nn // _RBS),
        in_specs=[
            pl.BlockSpec((1, _RBS, KK), lambda b, i: (b, i, 0)),
            pl.BlockSpec((1, Nn, 128), lambda b, i: (b, 0, 0)),
            pl.BlockSpec((1, Nn, 128), lambda b, i: (b, 0, 0)),
            pl.BlockSpec((1, _RBS, 128), lambda b, i: (b, i, 0)),
            pl.BlockSpec((128, 128), lambda b, i: (0, 0)),
            pl.BlockSpec((128, 128), lambda b, i: (0, 0)),
            pl.BlockSpec((1, 128), lambda b, i: (0, 0)),
            pl.BlockSpec((1, 128), lambda b, i: (0, 0)),
            pl.BlockSpec((1, 128), lambda b, i: (0, 0)),
        ],
        out_specs=[
            pl.BlockSpec((1, _RBS, 128), lambda b, i: (b, i, 0)),
            pl.BlockSpec((1, _RBS, 128), lambda b, i: (b, i, 0)),
            pl.BlockSpec((1, _RBS, 128), lambda b, i: (b, i, 0)),
        ],
        out_shape=[
            jax.ShapeDtypeStruct((Bb, Nn, 128), F32),
            jax.ShapeDtypeStruct((Bb, Nn, 128), jnp.bfloat16),
            jax.ShapeDtypeStruct((Bb, Nn, 128), jnp.bfloat16),
        ],
    )(idx, xh, xl, xb, wlt, wrt, bl, bn_s, bn_t)


def _fmax_body(x_ref, o_ref):
    o_ref[0] = jnp.max(x_ref[0], axis=0, keepdims=True)


def _fmax(h2):
    return pl.pallas_call(
        _fmax_body,
        grid=(Bb,),
        in_specs=[pl.BlockSpec((1, Nn, 128), lambda b: (b, 0, 0))],
        out_specs=pl.BlockSpec((1, 1, 128), lambda b: (b, 0, 0)),
        out_shape=jax.ShapeDtypeStruct((Bb, 1, 128), F32),
    )(h2)


_RQ = 512


def _q_body(x_ref, fm_ref, fg_ref, w1a_ref, w1b_ref, w1c_ref, b1_ref,
            w2_ref, b2_ref, o_ref):
    c = (_dot(fm_ref[0], w1b_ref[...]) + _dot(fg_ref[0], w1c_ref[...])
         + b1_ref[...])
    h = jnp.maximum(_dot(x_ref[0], w1a_ref[...]) + c, 0.0)
    o_ref[0] = _dot(h, w2_ref[...]) + b2_ref[...]


def _qkern(h2, fm, fg, w1at, w1bt, w1ct, b1, w2t, b2):
    return pl.pallas_call(
        _q_body,
        grid=(Bb, Nn // _RQ),
        in_specs=[
            pl.BlockSpec((1, _RQ, 128), lambda b, i: (b, i, 0)),
            pl.BlockSpec((1, 1, 128), lambda b, i: (b, 0, 0)),
            pl.BlockSpec((1, 1, DFf), lambda b, i: (b, 0, 0)),
            pl.BlockSpec((128, 256), lambda b, i: (0, 0)),
            pl.BlockSpec((128, 256), lambda b, i: (0, 0)),
            pl.BlockSpec((DFf, 256), lambda b, i: (0, 0)),
            pl.BlockSpec((1, 256), lambda b, i: (0, 0)),
            pl.BlockSpec((256, 128), lambda b, i: (0, 0)),
            pl.BlockSpec((1, 128), lambda b, i: (0, 0)),
        ],
        out_specs=pl.BlockSpec((1, _RQ, 128), lambda b, i: (b, i, 0)),
        out_shape=jax.ShapeDtypeStruct((Bb, Nn, 128), F32),
    )(h2, fm, fg, w1at, w1bt, w1ct, b1, w2t, b2)


def _kqv_body(kp_ref, q_ref, p16_ref, wst_ref, bs_ref, w1_ref, b1_ref,
              w2_ref, b2_ref, wk_ref, bk_ref, wq_ref, bq_ref, wv_ref,
              bv_ref, wu_ref, val_ref, tab_ref, qry_ref, vt_ref):
    kp = kp_ref[0]
    q = q_ref[0]
    cat = jnp.concatenate([kp, q], axis=1)
    sc = _dot(cat, wst_ref[...]) + bs_ref[...]
    h = jnp.maximum(_dot(cat, w1_ref[...]) + b1_ref[...], 0.0)
    val = _dot(h, w2_ref[...]) + b2_ref[...] + sc
    val_ref[0] = val
    key = _dot(kp, wk_ref[...]) + bk_ref[...]
    u = _dot(p16_ref[0], wu_ref[...])
    tab_ref[0] = jnp.concatenate([key, u], axis=1)
    qry_ref[0] = _dot(q, wq_ref[...]) + bq_ref[...]
    vt_ref[0] = _dot(val, wv_ref[...]) + bv_ref[...]


def _kqv(kp, q, p16, wst, bs, w1t, b1, w2t, b2, wkt, bk, wqt, bq, wvt, bv,
         wut):
    wspec = lambda s: pl.BlockSpec(s, lambda b, i: (0, 0))
    return pl.pallas_call(
        _kqv_body,
        grid=(Bb, Nn // _RQ),
        in_specs=[
            pl.BlockSpec((1, _RQ, 128), lambda b, i: (b, i, 0)),
            pl.BlockSpec((1, _RQ, 128), lambda b, i: (b, i, 0)),
            pl.BlockSpec((1, _RQ, 16), lambda b, i: (b, i, 0)),
            wspec((256, 128)), wspec((1, 128)),
            wspec((256, 128)), wspec((1, 128)),
            wspec((128, 128)), wspec((1, 128)),
            wspec((128, 64)), wspec((1, 64)),
            wspec((128, 64)), wspec((1, 64)),
            wspec((128, 64)), wspec((1, 64)),
            wspec((16, 64)),
        ],
        out_specs=[
            pl.BlockSpec((1, _RQ, 128), lambda b, i: (b, i, 0)),
            pl.BlockSpec((1, _RQ, 128), lambda b, i: (b, i, 0)),
            pl.BlockSpec((1, _RQ, 64), lambda b, i: (b, i, 0)),
            pl.BlockSpec((1, _RQ, 64), lambda b, i: (b, i, 0)),
        ],
        out_shape=[
            jax.ShapeDtypeStruct((Bb, Nn, 128), F32),
            jax.ShapeDtypeStruct((Bb, Nn, 128), F32),
            jax.ShapeDtypeStruct((Bb, Nn, 64), F32),
            jax.ShapeDtypeStruct((Bb, Nn, 64), F32),
        ],
    )(kp, q, p16, wst, bs, w1t, b1, w2t, b2, wkt, bk, wqt, bq, wvt, bv, wut)


_GCH = 128


def _sc_gather(table, ids):
    E = ids.shape[0]
    D = table.shape[1]
    info = plsc.get_sparse_core_info()
    nw = info.num_cores * info.num_subcores
    e_per_w = E // nw
    n_ch = e_per_w // _GCH
    mesh = plsc.VectorSubcoreMesh(core_axis_name="c", subcore_axis_name="s")

    @functools.partial(
        pl.kernel, mesh=mesh,
        out_type=jax.ShapeDtypeStruct((E, D), F32),
        scratch_types=[
            pltpu.VMEM((_GCH,), jnp.int32),
            pltpu.VMEM((_GCH, D), F32),
            pltpu.SemaphoreType.DMA,
        ],
    )
    def k(table_hbm, ids_hbm, out_hbm, idx_v, rows_v, sem):
        wid = lax.axis_index("s") * info.num_cores + lax.axis_index("c")
        base = wid * e_per_w

        def body(c, _):
            off = base + c * _GCH
            pltpu.sync_copy(ids_hbm.at[pl.ds(off, _GCH)], idx_v)
            pltpu.async_copy(table_hbm.at[idx_v], rows_v, sem).wait()
            pltpu.sync_copy(rows_v, out_hbm.at[pl.ds(off, _GCH)])
            return _

        lax.fori_loop(0, n_ch, body, 0)

    return k(table, ids)


_RA = 128


def _rep(x, c):
    return jnp.broadcast_to(x[:, None, :], (_RA, KK, c)).reshape(_RA * KK, c)


def _seg(x, c):
    return jnp.sum(x.reshape(_RA, KK, c), axis=1)


def _att_body(g_ref, q_ref, v_ref, tab_ref, idn_ref,
              bp1_ref, ps_ref, pt_ref, wp2_ref, bp2_ref,
              wa1_ref, ba1_ref, as_ref, at_ref, wa2_ref, ba2_ref,
              we_ref, be_ref, wf1_ref, bf1_ref, wf2_ref, bf2_ref,
              pw_ref, h_ref, y_ref):
    g = g_ref[0]
    qk = _rep(q_ref[0], 64) - g[:, :64]
    pr = _rep(tab_ref[0][:, 64:128], 64) - g[:, 64:128]
    pe = jnp.maximum((pr + bp1_ref[...]) * ps_ref[...] + pt_ref[...], 0.0)
    pe = _dot(pe, wp2_ref[...]) + bp2_ref[...]
    a = _dot(qk + pe, wa1_ref[...]) + ba1_ref[...]
    a = jnp.maximum(a * as_ref[...] + at_ref[...], 0.0)
    logit = _dot(a, wa2_ref[...]) + ba2_ref[...]
    e = jnp.exp(logit - jnp.max(logit))
    val4 = _rep(v_ref[0], 64) + pe
    numer = _seg(e * val4, 64)
    denom = _seg(e, 64)
    agg = numer / denom
    h = _dot(agg, we_ref[...]) + be_ref[...] + idn_ref[0]
    h_ref[0] = h
    fc = jnp.maximum(_dot(h, wf1_ref[...]) + bf1_ref[...], 0.0)
    fc = _dot(fc, wf2_ref[...]) + bf2_ref[...]
    y = _dot(fc, pw_ref[...])
    y_ref[0] = y.reshape(2 * _RA, 128)


def _att(g, qt, vt, tab, idn, bp1, pbs, pbt, wp2t, bp2,
         wa1t, ba1, abs_, abt, wa2t, ba2, wet, be, wf1t, bf1, wf2t, bf2,
         pw):
    wspec = lambda s: pl.BlockSpec(s, lambda b, i: (0, 0))
    return pl.pallas_call(
        _att_body,
        grid=(Bb, Nn // _RA),
        in_specs=[
            pl.BlockSpec((1, _RA * KK, 128), lambda b, i: (b, i, 0)),
            pl.BlockSpec((1, _RA, 64), lambda b, i: (b, i, 0)),
            pl.BlockSpec((1, _RA, 64), lambda b, i: (b, i, 0)),
            pl.BlockSpec((1, _RA, 128), lambda b, i: (b, i, 0)),
            pl.BlockSpec((1, _RA, 128), lambda b, i: (b, i, 0)),
            wspec((1, 64)), wspec((1, 64)), wspec((1, 64)),
            wspec((64, 64)), wspec((1, 64)),
            wspec((64, 256)), wspec((1, 256)), wspec((1, 256)), wspec((1, 256)),
            wspec((256, 64)), wspec((1, 64)),
            wspec((64, 128)), wspec((1, 128)),
            wspec((128, 64)), wspec((1, 64)),
            wspec((64, 32)), wspec((1, 32)),
            wspec((32, 256)),
        ],
        out_specs=[
            pl.BlockSpec((1, _RA, 128), lambda b, i: (b, i, 0)),
            pl.BlockSpec((1, 2 * _RA, 128), lambda b, i: (b, i, 0)),
        ],
        out_shape=[
            jax.ShapeDtypeStruct((Bb, Nn, 128), F32),
            jax.ShapeDtypeStruct((Bb, 2 * Nn, 128), F32),
        ],
    )(g, qt, vt, tab, idn, bp1, pbs, pbt, wp2t, bp2,
      wa1t, ba1, abs_, abt, wa2t, ba2, wet, be, wf1t, bf1, wf2t, bf2, pw)


_RF = 512


def _fin_body(fc_ref, h_ref, pts_ref, wsa_ref, wsb_ref, bs_ref,
              w1a_ref, w1b_ref, b1_ref, w2_ref, b2_ref,
              wd1_ref, bd1_ref, wd2_ref, bd2_ref, kc_ref, po_ref):
    fcx = fc_ref[0]
    hh = jnp.broadcast_to(h_ref[0][:, None, :], (_RF // 2, 2, 128)
                          ).reshape(_RF, 128)
    sc = _dot(fcx, wsa_ref[...]) + _dot(hh, wsb_ref[...]) + bs_ref[...]
    h = jnp.maximum(_dot(fcx, w1a_ref[...]) + _dot(hh, w1b_ref[...])
                    + b1_ref[...], 0.0)
    kc = _dot(h, w2_ref[...]) + b2_ref[...] + sc
    kc_ref[0] = kc
    r = jnp.maximum(kc, 0.0)
    d1 = jnp.maximum(_dot(r, wd1_ref[...]) + bd1_ref[...], 0.0)
    dl = _dot(d1, wd2_ref[...]) + bd2_ref[...]
    pp = jnp.broadcast_to(pts_ref[0][:, None, :], (_RF // 2, 2, 8)
                          ).reshape(_RF, 8)
    po_ref[0] = pp + jnp.tanh(dl)


def _fin(fc_rows, H, pts8, wsat, wsbt, bs, w1at, w1bt, b1, w2t, b2,
         wd1t, bd1, wd2t, bd2):
    wspec = lambda s: pl.BlockSpec(s, lambda b, i: (0, 0))
    N2 = 2 * Nn
    return pl.pallas_call(
        _fin_body,
        grid=(Bb, N2 // _RF),
        in_specs=[
            pl.BlockSpec((1, _RF, 128), lambda b, i: (b, i, 0)),
            pl.BlockSpec((1, _RF // 2, 128), lambda b, i: (b, i, 0)),
            pl.BlockSpec((1, _RF // 2, 8), lambda b, i: (b, i, 0)),
            wspec((128, 128)), wspec((128, 128)), wspec((1, 128)),
            wspec((128, 128)), wspec((128, 128)), wspec((1, 128)),
            wspec((128, 128)), wspec((1, 128)),
            wspec((128, 64)), wspec((1, 64)),
            wspec((64, 8)), wspec((1, 8)),
        ],
        out_specs=[
            pl.BlockSpec((1, _RF, 128), lambda b, i: (b, i, 0)),
            pl.BlockSpec((1, _RF, 8), lambda b, i: (b, i, 0)),
        ],
        out_shape=[
            jax.ShapeDtypeStruct((Bb, N2, 128), F32),
            jax.ShapeDtypeStruct((Bb, N2, 8), F32),
        ],
    )(fc_rows, H, pts8, wsat, wsbt, bs, w1at, w1bt, b1, w2t, b2,
      wd1t, bd1, wd2t, bd2)


def _bn_fold(p, pre):
    s = p[pre + '_g'] / jnp.sqrt(p[pre + '_v'] + EPS)
    t = p[pre + '_b'] - p[pre + '_m'] * s
    return s[None, :], t[None, :]


def _bn_fold2(p, pre):
    s = p[pre + 'g'] / jnp.sqrt(p[pre + 'v'] + EPS)
    t = p[pre + 'b'] - p[pre + 'm'] * s
    return s[None, :], t[None, :]


def _row(v):
    return v[None, :]


def kernel(pcd_prev, feat_global, K_prev, params):
    p = params
    pts = jnp.transpose(pcd_prev, (0, 2, 1))
    pts8 = jnp.pad(pts, ((0, 0), (0, 0), (0, 5)))
    pts16 = jnp.pad(pts, ((0, 0), (0, 0), (0, 13)))

    idx, idx2 = _knn(pts8)

    x, xh, xl = _mlp1(pts8.reshape(Bb * Nn, 8),
                      jnp.pad(p['mlp1_W1'].T, ((0, 5), (0, 0))),
                      _row(p['mlp1_b1']),
                      p['mlp1_W2'].T, _row(p['mlp1_b2']))
    x = x.reshape(Bb, Nn, 128)
    xh = xh.reshape(Bb, Nn, 128)
    xl = xl.reshape(Bb, Nn, 128)

    s1, t1 = _bn_fold(p, 'bn1')
    h1, h1h, h1l = _sage(xh, xl, x, idx, p['sage1_Wl'].T, _row(p['sage1_bl']),
                         p['sage1_Wr'].T, s1, t1, residual=False,
                         emit_split=True)
    s2, t2 = _bn_fold(p, 'bn2')
    h2, _, _ = _sage(h1h, h1l, h1, idx, p['sage2_Wl'].T, _row(p['sage2_bl']),
                     p['sage2_Wr'].T, s2, t2, residual=True,
                     emit_split=False)

    fm = _fmax(h2)
    fg = jnp.transpose(feat_global, (0, 2, 1))
    W1 = p['mlp2_W1']
    Q = _qkern(h2, fm, fg,
               W1[:, :128].T, W1[:, 128:256].T, W1[:, 256:].T,
               _row(p['mlp2_b1']), p['mlp2_W2'].T, _row(p['mlp2_b2']))

    kp = jnp.transpose(K_prev, (0, 2, 1))
    value, tab, query_t, value_t = _kqv(
        kp, Q, pts16,
        p['st_mlpv_Ws'].T, _row(p['st_mlpv_bs']),
        p['st_mlpv_W1'].T, _row(p['st_mlpv_b1']),
        p['st_mlpv_W2'].T, _row(p['st_mlpv_b2']),
        p['st_Wk'].T, _row(p['st_bk']),
        p['st_Wq'].T, _row(p['st_bq']),
        p['st_Wv'].T, _row(p['st_bv']),
        jnp.pad(p['st_pos_W1'].T, ((0, 13), (0, 0))))

    flat_ids = (idx2 + (jnp.arange(Bb, dtype=jnp.int32) * Nn)[:, None, None]
                ).reshape(-1)
    g = _sc_gather(tab.reshape(Bb * Nn, 128), flat_ids
                   ).reshape(Bb, Nn * KK, 128)

    pbs, pbt = _bn_fold2(p, 'st_pos_bn')
    abs_, abt = _bn_fold2(p, 'st_att_bn')
    H, fc_rows = _att(
        g, query_t, value_t, tab, value,
        _row(p['st_pos_b1']),
        pbs, pbt, p['st_pos_W2'].T, _row(p['st_pos_b2']),
        p['st_att_W1'].T, _row(p['st_att_b1']), abs_, abt,
        p['st_att_W2'].T, _row(p['st_att_b2']),
        p['st_We'].T, _row(p['st_be']),
        p['mlpps_W1'].T, _row(p['mlpps_b1']),
        p['mlpps_W2'].T, _row(p['mlpps_b2']),
        jnp.transpose(p['ps_W'], (0, 2, 1)).reshape(32, 256))

    Wdfs = p['mlpdf_Ws'].T
    Wdf1 = p['mlpdf_W1'].T
    kc, po = _fin(fc_rows, H, pts8,
                  Wdfs[:128], Wdfs[128:], _row(p['mlpdf_bs']),
                  Wdf1[:128], Wdf1[128:], _row(p['mlpdf_b1']),
                  p['mlpdf_W2'].T, _row(p['mlpdf_b2']),
                  p['mlpd_W1'].T, _row(p['mlpd_b1']),
                  jnp.pad(p['mlpd_W2'].T, ((0, 0), (0, 5))),
                  jnp.pad(_row(p['mlpd_b2']), ((0, 0), (0, 5))))

    pcd_child = jnp.transpose(po[:, :, :3], (0, 2, 1))
    K_curr = jnp.transpose(kc, (0, 2, 1))
    return pcd_child, K_curr


# speedup vs baseline: 1.1843x; 1.1843x over previous
import functools

import jax
import jax.numpy as jnp
from jax import lax
from jax.experimental import pallas as pl
from jax.experimental.pallas import tpu as pltpu
from jax.experimental.pallas import tpu_sc as plsc

Bb, Nn, DFf, KK = 4, 2048, 512, 16
EPS = 1e-5
F32 = jnp.float32


def _hi(x):
    return x


def _dotT(a, b):
    return lax.dot_general(a, b, (((1,), (1,)), ((), ())),
                           preferred_element_type=F32)


def _dot(a, b):
    return lax.dot_general(a, b, (((1,), (0,)), ((), ())),
                           preferred_element_type=F32)


_RBK = 256


def _knn_body(ptsb_ref, ptsf_ref, ones_ref, idx_ref, idx2_ref):
    pb = ptsb_ref[0]
    pf = ptsf_ref[0]
    sq_b = jnp.sum(pb * pb, axis=1, keepdims=True)
    sq_f = _dotT(ones_ref[...], pf * pf)
    g = _dotT(pb, pf)
    d = sq_b + sq_f - 2.0 * g
    col = lax.broadcasted_iota(jnp.int32, (_RBK, Nn), 1)
    big = jnp.float32(1e30)
    i = pl.program_id(1)
    n_self = i * _RBK + lax.broadcasted_iota(jnp.int32, (_RBK, 1), 0)
    dd = jnp.where(col == n_self, big, d)
    colf = col.astype(F32)
    ams = []
    for _ in range(KK):
        dmin = jnp.min(dd, axis=1, keepdims=True)
        eqm = dd == dmin
        am = jnp.min(jnp.where(eqm, colf, jnp.float32(Nn)), axis=1,
                     keepdims=True)
        ams.append(am)
        dd = jnp.where(eqm, big, dd)
    v = jnp.concatenate(ams, axis=1).astype(jnp.int32)
    idx_ref[0] = v
    idx2_ref[0] = jnp.concatenate([n_self, v[:, :KK - 1]], axis=1)


def _knn(pts8):
    grid = (Bb, Nn // _RBK)
    out = pl.pallas_call(
        _knn_body,
        grid=grid,
        in_specs=[
            pl.BlockSpec((1, _RBK, 8), lambda b, i: (b, i, 0)),
            pl.BlockSpec((1, Nn, 8), lambda b, i: (b, 0, 0)),
            pl.BlockSpec((1, 8), lambda b, i: (0, 0)),
        ],
        out_specs=[
            pl.BlockSpec((1, _RBK, KK), lambda b, i: (b, i, 0)),
            pl.BlockSpec((1, _RBK, KK), lambda b, i: (b, i, 0)),
        ],
        out_shape=[
            jax.ShapeDtypeStruct((Bb, Nn, KK), jnp.int32),
            jax.ShapeDtypeStruct((Bb, Nn, KK), jnp.int32),
        ],
    )(pts8, pts8, jnp.ones((1, 8), F32))
    return out


def _mlp1_body(x_ref, w1_ref, b1_ref, w2_ref, b2_ref, o_ref):
    h = jnp.maximum(_dot(x_ref[...], w1_ref[...]) + b1_ref[...], 0.0)
    o_ref[...] = _dot(h, w2_ref[...]) + b2_ref[...]


def _mlp1(xrows, w1t, b1, w2t, b2):
    R = xrows.shape[0]
    RB = 1024
    return pl.pallas_call(
        _mlp1_body,
        grid=(R // RB,),
        in_specs=[
            pl.BlockSpec((RB, 8), lambda i: (i, 0)),
            pl.BlockSpec(w1t.shape, lambda i: (0, 0)),
            pl.BlockSpec(b1.shape, lambda i: (0, 0)),
            pl.BlockSpec(w2t.shape, lambda i: (0, 0)),
            pl.BlockSpec(b2.shape, lambda i: (0, 0)),
        ],
        out_specs=pl.BlockSpec((RB, 128), lambda i: (i, 0)),
        out_shape=jax.ShapeDtypeStruct((R, 128), F32),
    )(xrows, w1t, b1, w2t, b2)


_RBS = 256


def _sage_body(idx_ref, xf_ref, xb_ref, wl_ref, wr_ref, cb_ref, sc_ref,
               sh_ref, o_ref, *, residual):
    idx = idx_ref[0]
    xf = xf_ref[0]
    xb = xb_ref[0]
    col = lax.broadcasted_iota(jnp.int32, (_RBS, Nn), 1)
    acc = jnp.zeros((_RBS, Nn), F32)
    for k in range(KK):
        acc = acc + (col == idx[:, k:k + 1]).astype(F32)
    mean = _dot(acc, xf) * (1.0 / KK)
    y = _dot(mean, wl_ref[...]) + cb_ref[...] + _dot(xb, wr_ref[...])
    y = jnp.maximum(y * sc_ref[...] + sh_ref[...], 0.0)
    if residual:
        y = y + xb
    o_ref[0] = y


def _sage(x, idx, wlt, bl, wrt, bn_s, bn_t, residual):
    body = functools.partial(_sage_body, residual=residual)
    return pl.pallas_call(
        body,
        grid=(Bb, Nn // _RBS),
        in_specs=[
            pl.BlockSpec((1, _RBS, KK), lambda b, i: (b, i, 0)),
            pl.BlockSpec((1, Nn, 128), lambda b, i: (b, 0, 0)),
            pl.BlockSpec((1, _RBS, 128), lambda b, i: (b, i, 0)),
            pl.BlockSpec((128, 128), lambda b, i: (0, 0)),
            pl.BlockSpec((128, 128), lambda b, i: (0, 0)),
            pl.BlockSpec((1, 128), lambda b, i: (0, 0)),
            pl.BlockSpec((1, 128), lambda b, i: (0, 0)),
            pl.BlockSpec((1, 128), lambda b, i: (0, 0)),
        ],
        out_specs=pl.BlockSpec((1, _RBS, 128), lambda b, i: (b, i, 0)),
        out_shape=jax.ShapeDtypeStruct((Bb, Nn, 128), F32),
    )(idx, x, x, wlt, wrt, bl, bn_s, bn_t)


def _fmax_body(x_ref, o_ref):
    o_ref[0] = jnp.max(x_ref[0], axis=0, keepdims=True)


def _fmax(h2):
    return pl.pallas_call(
        _fmax_body,
        grid=(Bb,),
        in_specs=[pl.BlockSpec((1, Nn, 128), lambda b: (b, 0, 0))],
        out_specs=pl.BlockSpec((1, 1, 128), lambda b: (b, 0, 0)),
        out_shape=jax.ShapeDtypeStruct((Bb, 1, 128), F32),
    )(h2)


_RQ = 512


def _q_body(x_ref, fm_ref, fg_ref, w1a_ref, w1b_ref, w1c_ref, b1_ref,
            w2_ref, b2_ref, o_ref):
    c = (_dot(fm_ref[0], w1b_ref[...]) + _dot(fg_ref[0], w1c_ref[...])
         + b1_ref[...])
    h = jnp.maximum(_dot(x_ref[0], w1a_ref[...]) + c, 0.0)
    o_ref[0] = _dot(h, w2_ref[...]) + b2_ref[...]


def _qkern(h2, fm, fg, w1at, w1bt, w1ct, b1, w2t, b2):
    return pl.pallas_call(
        _q_body,
        grid=(Bb, Nn // _RQ),
        in_specs=[
            pl.BlockSpec((1, _RQ, 128), lambda b, i: (b, i, 0)),
            pl.BlockSpec((1, 1, 128), lambda b, i: (b, 0, 0)),
            pl.BlockSpec((1, 1, DFf), lambda b, i: (b, 0, 0)),
            pl.BlockSpec((128, 256), lambda b, i: (0, 0)),
            pl.BlockSpec((128, 256), lambda b, i: (0, 0)),
            pl.BlockSpec((DFf, 256), lambda b, i: (0, 0)),
            pl.BlockSpec((1, 256), lambda b, i: (0, 0)),
            pl.BlockSpec((256, 128), lambda b, i: (0, 0)),
            pl.BlockSpec((1, 128), lambda b, i: (0, 0)),
        ],
        out_specs=pl.BlockSpec((1, _RQ, 128), lambda b, i: (b, i, 0)),
        out_shape=jax.ShapeDtypeStruct((Bb, Nn, 128), F32),
    )(h2, fm, fg, w1at, w1bt, w1ct, b1, w2t, b2)


def _kqv_body(kp_ref, q_ref, p16_ref, wst_ref, bs_ref, w1_ref, b1_ref,
              w2_ref, b2_ref, wk_ref, bk_ref, wq_ref, bq_ref, wv_ref,
              bv_ref, wu_ref, val_ref, tab_ref, qry_ref, vt_ref):
    kp = jnp.transpose(kp_ref[0])
    q = q_ref[0]
    cat = jnp.concatenate([kp, q], axis=1)
    sc = _dot(cat, wst_ref[...]) + bs_ref[...]
    h = jnp.maximum(_dot(cat, w1_ref[...]) + b1_ref[...], 0.0)
    val = _dot(h, w2_ref[...]) + b2_ref[...] + sc
    val_ref[0] = val
    key = _dot(kp, wk_ref[...]) + bk_ref[...]
    u = _dot(p16_ref[0], wu_ref[...])
    tab_ref[0] = jnp.concatenate([key, u], axis=1)
    qry_ref[0] = _dot(q, wq_ref[...]) + bq_ref[...]
    vt_ref[0] = _dot(val, wv_ref[...]) + bv_ref[...]


def _kqv(kp, q, p16, wst, bs, w1t, b1, w2t, b2, wkt, bk, wqt, bq, wvt, bv,
         wut):
    wspec = lambda s: pl.BlockSpec(s, lambda b, i: (0, 0))
    return pl.pallas_call(
        _kqv_body,
        grid=(Bb, Nn // _RQ),
        in_specs=[
            pl.BlockSpec((1, 128, _RQ), lambda b, i: (b, 0, i)),
            pl.BlockSpec((1, _RQ, 128), lambda b, i: (b, i, 0)),
            pl.BlockSpec((1, _RQ, 16), lambda b, i: (b, i, 0)),
            wspec((256, 128)), wspec((1, 128)),
            wspec((256, 128)), wspec((1, 128)),
            wspec((128, 128)), wspec((1, 128)),
            wspec((128, 64)), wspec((1, 64)),
            wspec((128, 64)), wspec((1, 64)),
            wspec((128, 64)), wspec((1, 64)),
            wspec((16, 64)),
        ],
        out_specs=[
            pl.BlockSpec((1, _RQ, 128), lambda b, i: (b, i, 0)),
            pl.BlockSpec((1, _RQ, 128), lambda b, i: (b, i, 0)),
            pl.BlockSpec((1, _RQ, 64), lambda b, i: (b, i, 0)),
            pl.BlockSpec((1, _RQ, 64), lambda b, i: (b, i, 0)),
        ],
        out_shape=[
            jax.ShapeDtypeStruct((Bb, Nn, 128), F32),
            jax.ShapeDtypeStruct((Bb, Nn, 128), F32),
            jax.ShapeDtypeStruct((Bb, Nn, 64), F32),
            jax.ShapeDtypeStruct((Bb, Nn, 64), F32),
        ],
    )(kp, q, p16, wst, bs, w1t, b1, w2t, b2, wkt, bk, wqt, bq, wvt, bv, wut)


_GCH = 128


def _sc_gather(table, ids):
    E = ids.shape[0]
    D = table.shape[1]
    info = plsc.get_sparse_core_info()
    nw = info.num_cores * info.num_subcores
    e_per_w = E // nw
    n_ch = e_per_w // _GCH
    mesh = plsc.VectorSubcoreMesh(core_axis_name="c", subcore_axis_name="s")

    @functools.partial(
        pl.kernel, mesh=mesh,
        out_type=jax.ShapeDtypeStruct((E, D), F32),
        scratch_types=[
            pltpu.VMEM((_GCH,), jnp.int32),
            pltpu.VMEM((_GCH, D), F32),
            pltpu.SemaphoreType.DMA,
        ],
    )
    def k(table_hbm, ids_hbm, out_hbm, idx_v, rows_v, sem):
        wid = lax.axis_index("s") * info.num_cores + lax.axis_index("c")
        base = wid * e_per_w

        def body(c, _):
            off = base + c * _GCH
            pltpu.sync_copy(ids_hbm.at[pl.ds(off, _GCH)], idx_v)
            pltpu.async_copy(table_hbm.at[idx_v], rows_v, sem).wait()
            pltpu.sync_copy(rows_v, out_hbm.at[pl.ds(off, _GCH)])
            return _

        lax.fori_loop(0, n_ch, body, 0)

    return k(table, ids)


_RA = 128


def _rep(x, c):
    return jnp.broadcast_to(x[:, None, :], (_RA, KK, c)).reshape(_RA * KK, c)


def _seg(x, c):
    return jnp.sum(x.reshape(_RA, KK, c), axis=1)


def _att_body(g_ref, q_ref, v_ref, tab_ref, idn_ref,
              bp1_ref, ps_ref, pt_ref, wp2_ref, bp2_ref,
              wa1_ref, ba1_ref, as_ref, at_ref, wa2_ref, ba2_ref,
              we_ref, be_ref, wf1_ref, bf1_ref, wf2_ref, bf2_ref,
              pw_ref, h_ref, y_ref):
    g = g_ref[0]
    qk = _rep(q_ref[0], 64) - g[:, :64]
    pr = _rep(tab_ref[0][:, 64:128], 64) - g[:, 64:128]
    pe = jnp.maximum((pr + bp1_ref[...]) * ps_ref[...] + pt_ref[...], 0.0)
    pe = _dot(pe, wp2_ref[...]) + bp2_ref[...]
    a = _dot(qk + pe, wa1_ref[...]) + ba1_ref[...]
    a = jnp.maximum(a * as_ref[...] + at_ref[...], 0.0)
    logit = _dot(a, wa2_ref[...]) + ba2_ref[...]
    e = jnp.exp(logit - jnp.max(logit))
    val4 = _rep(v_ref[0], 64) + pe
    numer = _seg(e * val4, 64)
    denom = _seg(e, 64)
    agg = numer / denom
    h = _dot(agg, we_ref[...]) + be_ref[...] + idn_ref[0]
    h_ref[0] = h
    fc = jnp.maximum(_dot(h, wf1_ref[...]) + bf1_ref[...], 0.0)
    fc = _dot(fc, wf2_ref[...]) + bf2_ref[...]
    y = _dot(fc, pw_ref[...])
    y_ref[0] = y.reshape(2 * _RA, 128)


def _att(g, qt, vt, tab, idn, bp1, pbs, pbt, wp2t, bp2,
         wa1t, ba1, abs_, abt, wa2t, ba2, wet, be, wf1t, bf1, wf2t, bf2,
         pw):
    wspec = lambda s: pl.BlockSpec(s, lambda b, i: (0, 0))
    return pl.pallas_call(
        _att_body,
        grid=(Bb, Nn // _RA),
        in_specs=[
            pl.BlockSpec((1, _RA * KK, 128), lambda b, i: (b, i, 0)),
            pl.BlockSpec((1, _RA, 64), lambda b, i: (b, i, 0)),
            pl.BlockSpec((1, _RA, 64), lambda b, i: (b, i, 0)),
            pl.BlockSpec((1, _RA, 128), lambda b, i: (b, i, 0)),
            pl.BlockSpec((1, _RA, 128), lambda b, i: (b, i, 0)),
            wspec((1, 64)), wspec((1, 64)), wspec((1, 64)),
            wspec((64, 64)), wspec((1, 64)),
            wspec((64, 256)), wspec((1, 256)), wspec((1, 256)), wspec((1, 256)),
            wspec((256, 64)), wspec((1, 64)),
            wspec((64, 128)), wspec((1, 128)),
            wspec((128, 64)), wspec((1, 64)),
            wspec((64, 32)), wspec((1, 32)),
            wspec((32, 256)),
        ],
        out_specs=[
            pl.BlockSpec((1, _RA, 128), lambda b, i: (b, i, 0)),
            pl.BlockSpec((1, 2 * _RA, 128), lambda b, i: (b, i, 0)),
        ],
        out_shape=[
            jax.ShapeDtypeStruct((Bb, Nn, 128), F32),
            jax.ShapeDtypeStruct((Bb, 2 * Nn, 128), F32),
        ],
    )(g, qt, vt, tab, idn, bp1, pbs, pbt, wp2t, bp2,
      wa1t, ba1, abs_, abt, wa2t, ba2, wet, be, wf1t, bf1, wf2t, bf2, pw)


_RF = 512


def _fin_body(fc_ref, h_ref, pts_ref, wsa_ref, wsb_ref, bs_ref,
              w1a_ref, w1b_ref, b1_ref, w2_ref, b2_ref,
              wd1_ref, bd1_ref, wd2_ref, bd2_ref, kc_ref, po_ref):
    fcx = fc_ref[0]
    hh = jnp.broadcast_to(h_ref[0][:, None, :], (_RF // 2, 2, 128)
                          ).reshape(_RF, 128)
    sc = _dot(fcx, wsa_ref[...]) + _dot(hh, wsb_ref[...]) + bs_ref[...]
    h = jnp.maximum(_dot(fcx, w1a_ref[...]) + _dot(hh, w1b_ref[...])
                    + b1_ref[...], 0.0)
    kc = _dot(h, w2_ref[...]) + b2_ref[...] + sc
    kc_ref[0] = jnp.transpose(kc)
    r = jnp.maximum(kc, 0.0)
    d1 = jnp.maximum(_dot(r, wd1_ref[...]) + bd1_ref[...], 0.0)
    dl = _dot(d1, wd2_ref[...]) + bd2_ref[...]
    pp = jnp.broadcast_to(pts_ref[0][:, None, :], (_RF // 2, 2, 8)
                          ).reshape(_RF, 8)
    po_ref[0] = jnp.transpose(pp + jnp.tanh(dl))


def _fin(fc_rows, H, pts8, wsat, wsbt, bs, w1at, w1bt, b1, w2t, b2,
         wd1t, bd1, wd2t, bd2):
    wspec = lambda s: pl.BlockSpec(s, lambda b, i: (0, 0))
    N2 = 2 * Nn
    return pl.pallas_call(
        _fin_body,
        grid=(Bb, N2 // _RF),
        in_specs=[
            pl.BlockSpec((1, _RF, 128), lambda b, i: (b, i, 0)),
            pl.BlockSpec((1, _RF // 2, 128), lambda b, i: (b, i, 0)),
            pl.BlockSpec((1, _RF // 2, 8), lambda b, i: (b, i, 0)),
            wspec((128, 128)), wspec((128, 128)), wspec((1, 128)),
            wspec((128, 128)), wspec((128, 128)), wspec((1, 128)),
            wspec((128, 128)), wspec((1, 128)),
            wspec((128, 64)), wspec((1, 64)),
            wspec((64, 8)), wspec((1, 8)),
        ],
        out_specs=[
            pl.BlockSpec((1, 128, _RF), lambda b, i: (b, 0, i)),
            pl.BlockSpec((1, 8, _RF), lambda b, i: (b, 0, i)),
        ],
        out_shape=[
            jax.ShapeDtypeStruct((Bb, 128, N2), F32),
            jax.ShapeDtypeStruct((Bb, 8, N2), F32),
        ],
    )(fc_rows, H, pts8, wsat, wsbt, bs, w1at, w1bt, b1, w2t, b2,
      wd1t, bd1, wd2t, bd2)


def _bn_fold(p, pre):
    s = p[pre + '_g'] / jnp.sqrt(p[pre + '_v'] + EPS)
    t = p[pre + '_b'] - p[pre + '_m'] * s
    return s[None, :], t[None, :]


def _bn_fold2(p, pre):
    s = p[pre + 'g'] / jnp.sqrt(p[pre + 'v'] + EPS)
    t = p[pre + 'b'] - p[pre + 'm'] * s
    return s[None, :], t[None, :]


def _row(v):
    return v[None, :]


def kernel(pcd_prev, feat_global, K_prev, params):
    p = params
    pts = jnp.transpose(pcd_prev, (0, 2, 1))
    pts8 = jnp.pad(pts, ((0, 0), (0, 0), (0, 5)))
    pts16 = jnp.pad(pts, ((0, 0), (0, 0), (0, 13)))

    idx, idx2 = _knn(pts8)

    x = _mlp1(pts8.reshape(Bb * Nn, 8),
              jnp.pad(p['mlp1_W1'].T, ((0, 5), (0, 0))), _row(p['mlp1_b1']),
              p['mlp1_W2'].T, _row(p['mlp1_b2'])).reshape(Bb, Nn, 128)

    s1, t1 = _bn_fold(p, 'bn1')
    h1 = _sage(x, idx, p['sage1_Wl'].T, _row(p['sage1_bl']),
               p['sage1_Wr'].T, s1, t1, residual=False)
    s2, t2 = _bn_fold(p, 'bn2')
    h2 = _sage(h1, idx, p['sage2_Wl'].T, _row(p['sage2_bl']),
               p['sage2_Wr'].T, s2, t2, residual=True)

    fm = _fmax(h2)
    fg = jnp.transpose(feat_global, (0, 2, 1))
    W1 = p['mlp2_W1']
    Q = _qkern(h2, fm, fg,
               W1[:, :128].T, W1[:, 128:256].T, W1[:, 256:].T,
               _row(p['mlp2_b1']), p['mlp2_W2'].T, _row(p['mlp2_b2']))

    value, tab, query_t, value_t = _kqv(
        K_prev, Q, pts16,
        p['st_mlpv_Ws'].T, _row(p['st_mlpv_bs']),
        p['st_mlpv_W1'].T, _row(p['st_mlpv_b1']),
        p['st_mlpv_W2'].T, _row(p['st_mlpv_b2']),
        p['st_Wk'].T, _row(p['st_bk']),
        p['st_Wq'].T, _row(p['st_bq']),
        p['st_Wv'].T, _row(p['st_bv']),
        jnp.pad(p['st_pos_W1'].T, ((0, 13), (0, 0))))

    flat_ids = (idx2 + (jnp.arange(Bb, dtype=jnp.int32) * Nn)[:, None, None]
                ).reshape(-1)
    g = _sc_gather(tab.reshape(Bb * Nn, 128), flat_ids
                   ).reshape(Bb, Nn * KK, 128)

    pbs, pbt = _bn_fold2(p, 'st_pos_bn')
    abs_, abt = _bn_fold2(p, 'st_att_bn')
    H, fc_rows = _att(
        g, query_t, value_t, tab, value,
        _row(p['st_pos_b1']),
        pbs, pbt, p['st_pos_W2'].T, _row(p['st_pos_b2']),
        p['st_att_W1'].T, _row(p['st_att_b1']), abs_, abt,
        p['st_att_W2'].T, _row(p['st_att_b2']),
        p['st_We'].T, _row(p['st_be']),
        p['mlpps_W1'].T, _row(p['mlpps_b1']),
        p['mlpps_W2'].T, _row(p['mlpps_b2']),
        jnp.transpose(p['ps_W'], (0, 2, 1)).reshape(32, 256))

    Wdfs = p['mlpdf_Ws'].T
    Wdf1 = p['mlpdf_W1'].T
    kc, po = _fin(fc_rows, H, pts8,
                  Wdfs[:128], Wdfs[128:], _row(p['mlpdf_bs']),
                  Wdf1[:128], Wdf1[128:], _row(p['mlpdf_b1']),
                  p['mlpdf_W2'].T, _row(p['mlpdf_b2']),
                  p['mlpd_W1'].T, _row(p['mlpd_b1']),
                  jnp.pad(p['mlpd_W2'].T, ((0, 0), (0, 5))),
                  jnp.pad(_row(p['mlpd_b2']), ((0, 0), (0, 5))))

    return po[:, :3, :], kc


# speedup vs baseline: 1.2405x; 1.0475x over previous
import functools

import jax
import jax.numpy as jnp
from jax import lax
from jax.experimental import pallas as pl
from jax.experimental.pallas import tpu as pltpu
from jax.experimental.pallas import tpu_sc as plsc

Bb, Nn, DFf, KK = 4, 2048, 512, 16
EPS = 1e-5
F32 = jnp.float32


def _hi(x):
    return x


def _dotT(a, b):
    return lax.dot_general(a, b, (((1,), (1,)), ((), ())),
                           preferred_element_type=F32)


def _dot(a, b):
    return lax.dot_general(a, b, (((1,), (0,)), ((), ())),
                           preferred_element_type=F32)


_RBK = 256


def _knn_body(ptsb_ref, ptsf_ref, ones_ref, idx_ref, idx2_ref):
    pb = ptsb_ref[0]
    pf = ptsf_ref[0]
    sq_b = jnp.sum(pb * pb, axis=1, keepdims=True)
    sq_f = _dotT(ones_ref[...], pf * pf)
    g = _dotT(pb, pf)
    d = sq_b + sq_f - 2.0 * g
    col = lax.broadcasted_iota(jnp.int32, (_RBK, Nn), 1)
    big = jnp.float32(1e30)
    i = pl.program_id(1)
    n_self = i * _RBK + lax.broadcasted_iota(jnp.int32, (_RBK, 1), 0)
    dd = jnp.where(col == n_self, big, d)
    colf = col.astype(F32)
    ams = []
    for _ in range(KK):
        dmin = jnp.min(dd, axis=1, keepdims=True)
        eqm = dd == dmin
        am = jnp.min(jnp.where(eqm, colf, jnp.float32(Nn)), axis=1,
                     keepdims=True)
        ams.append(am)
        dd = jnp.where(eqm, big, dd)
    v = jnp.concatenate(ams, axis=1).astype(jnp.int32)
    idx_ref[0] = v
    idx2_ref[0] = jnp.concatenate([n_self, v[:, :KK - 1]], axis=1)


def _knn(pts8):
    grid = (Bb, Nn // _RBK)
    out = pl.pallas_call(
        _knn_body,
        grid=grid,
        in_specs=[
            pl.BlockSpec((1, _RBK, 8), lambda b, i: (b, i, 0)),
            pl.BlockSpec((1, Nn, 8), lambda b, i: (b, 0, 0)),
            pl.BlockSpec((1, 8), lambda b, i: (0, 0)),
        ],
        out_specs=[
            pl.BlockSpec((1, _RBK, KK), lambda b, i: (b, i, 0)),
            pl.BlockSpec((1, _RBK, KK), lambda b, i: (b, i, 0)),
        ],
        out_shape=[
            jax.ShapeDtypeStruct((Bb, Nn, KK), jnp.int32),
            jax.ShapeDtypeStruct((Bb, Nn, KK), jnp.int32),
        ],
    )(pts8, pts8, jnp.ones((1, 8), F32))
    return out


def _mlp1_body(x_ref, w1_ref, b1_ref, w2_ref, b2_ref, o_ref):
    h = jnp.maximum(_dot(x_ref[...], w1_ref[...]) + b1_ref[...], 0.0)
    o_ref[...] = _dot(h, w2_ref[...]) + b2_ref[...]


def _mlp1(xrows, w1t, b1, w2t, b2):
    R = xrows.shape[0]
    RB = 1024
    return pl.pallas_call(
        _mlp1_body,
        grid=(R // RB,),
        in_specs=[
            pl.BlockSpec((RB, 8), lambda i: (i, 0)),
            pl.BlockSpec(w1t.shape, lambda i: (0, 0)),
            pl.BlockSpec(b1.shape, lambda i: (0, 0)),
            pl.BlockSpec(w2t.shape, lambda i: (0, 0)),
            pl.BlockSpec(b2.shape, lambda i: (0, 0)),
        ],
        out_specs=pl.BlockSpec((RB, 128), lambda i: (i, 0)),
        out_shape=jax.ShapeDtypeStruct((R, 128), F32),
    )(xrows, w1t, b1, w2t, b2)


_RBS = 256


def _sage_body(idx_ref, xf_ref, xb_ref, wl_ref, wr_ref, cb_ref, sc_ref,
               sh_ref, o_ref, *, residual):
    idx = idx_ref[0]
    xf = xf_ref[0]
    xb = xb_ref[0]
    col = lax.broadcasted_iota(jnp.int32, (_RBS, Nn), 1)
    acc = jnp.zeros((_RBS, Nn), F32)
    for k in range(KK):
        acc = acc + (col == idx[:, k:k + 1]).astype(F32)
    mean = _dot(acc, xf) * (1.0 / KK)
    y = _dot(mean, wl_ref[...]) + cb_ref[...] + _dot(xb, wr_ref[...])
    y = jnp.maximum(y * sc_ref[...] + sh_ref[...], 0.0)
    if residual:
        y = y + xb
    o_ref[0] = y


def _sage(x, idx, wlt, bl, wrt, bn_s, bn_t, residual):
    body = functools.partial(_sage_body, residual=residual)
    return pl.pallas_call(
        body,
        grid=(Bb, Nn // _RBS),
        in_specs=[
            pl.BlockSpec((1, _RBS, KK), lambda b, i: (b, i, 0)),
            pl.BlockSpec((1, Nn, 128), lambda b, i: (b, 0, 0)),
            pl.BlockSpec((1, _RBS, 128), lambda b, i: (b, i, 0)),
            pl.BlockSpec((128, 128), lambda b, i: (0, 0)),
            pl.BlockSpec((128, 128), lambda b, i: (0, 0)),
            pl.BlockSpec((1, 128), lambda b, i: (0, 0)),
            pl.BlockSpec((1, 128), lambda b, i: (0, 0)),
            pl.BlockSpec((1, 128), lambda b, i: (0, 0)),
        ],
        out_specs=pl.BlockSpec((1, _RBS, 128), lambda b, i: (b, i, 0)),
        out_shape=jax.ShapeDtypeStruct((Bb, Nn, 128), F32),
    )(idx, x, x, wlt, wrt, bl, bn_s, bn_t)


def _fmax_body(x_ref, o_ref):
    o_ref[0] = jnp.max(x_ref[0], axis=0, keepdims=True)


def _fmax(h2):
    return pl.pallas_call(
        _fmax_body,
        grid=(Bb,),
        in_specs=[pl.BlockSpec((1, Nn, 128), lambda b: (b, 0, 0))],
        out_specs=pl.BlockSpec((1, 1, 128), lambda b: (b, 0, 0)),
        out_shape=jax.ShapeDtypeStruct((Bb, 1, 128), F32),
    )(h2)


_RQ = 512


def _q_body(x_ref, fm_ref, fg_ref, w1a_ref, w1b_ref, w1c_ref, b1_ref,
            w2_ref, b2_ref, o_ref):
    c = (_dot(fm_ref[0], w1b_ref[...]) + _dot(fg_ref[0], w1c_ref[...])
         + b1_ref[...])
    h = jnp.maximum(_dot(x_ref[0], w1a_ref[...]) + c, 0.0)
    o_ref[0] = _dot(h, w2_ref[...]) + b2_ref[...]


def _qkern(h2, fm, fg, w1at, w1bt, w1ct, b1, w2t, b2):
    return pl.pallas_call(
        _q_body,
        grid=(Bb, Nn // _RQ),
        in_specs=[
            pl.BlockSpec((1, _RQ, 128), lambda b, i: (b, i, 0)),
            pl.BlockSpec((1, 1, 128), lambda b, i: (b, 0, 0)),
            pl.BlockSpec((1, 1, DFf), lambda b, i: (b, 0, 0)),
            pl.BlockSpec((128, 256), lambda b, i: (0, 0)),
            pl.BlockSpec((128, 256), lambda b, i: (0, 0)),
            pl.BlockSpec((DFf, 256), lambda b, i: (0, 0)),
            pl.BlockSpec((1, 256), lambda b, i: (0, 0)),
            pl.BlockSpec((256, 128), lambda b, i: (0, 0)),
            pl.BlockSpec((1, 128), lambda b, i: (0, 0)),
        ],
        out_specs=pl.BlockSpec((1, _RQ, 128), lambda b, i: (b, i, 0)),
        out_shape=jax.ShapeDtypeStruct((Bb, Nn, 128), F32),
    )(h2, fm, fg, w1at, w1bt, w1ct, b1, w2t, b2)


def _kqv_body(kp_ref, q_ref, p16_ref, wst_ref, bs_ref, w1_ref, b1_ref,
              w2_ref, b2_ref, wk_ref, bk_ref, wq_ref, bq_ref, wv_ref,
              bv_ref, wu_ref, val_ref, tab_ref, qry_ref, vt_ref):
    kp = jnp.transpose(kp_ref[0])
    q = q_ref[0]
    cat = jnp.concatenate([kp, q], axis=1)
    sc = _dot(cat, wst_ref[...]) + bs_ref[...]
    h = jnp.maximum(_dot(cat, w1_ref[...]) + b1_ref[...], 0.0)
    val = _dot(h, w2_ref[...]) + b2_ref[...] + sc
    val_ref[0] = val
    key = _dot(kp, wk_ref[...]) + bk_ref[...]
    u = _dot(p16_ref[0], wu_ref[...])
    tab_ref[0] = jnp.concatenate([key, u], axis=1)
    qry_ref[0] = _dot(q, wq_ref[...]) + bq_ref[...]
    vt_ref[0] = _dot(val, wv_ref[...]) + bv_ref[...]


def _kqv(kp, q, p16, wst, bs, w1t, b1, w2t, b2, wkt, bk, wqt, bq, wvt, bv,
         wut):
    wspec = lambda s: pl.BlockSpec(s, lambda b, i: (0, 0))
    return pl.pallas_call(
        _kqv_body,
        grid=(Bb, Nn // _RQ),
        in_specs=[
            pl.BlockSpec((1, 128, _RQ), lambda b, i: (b, 0, i)),
            pl.BlockSpec((1, _RQ, 128), lambda b, i: (b, i, 0)),
            pl.BlockSpec((1, _RQ, 16), lambda b, i: (b, i, 0)),
            wspec((256, 128)), wspec((1, 128)),
            wspec((256, 128)), wspec((1, 128)),
            wspec((128, 128)), wspec((1, 128)),
            wspec((128, 64)), wspec((1, 64)),
            wspec((128, 64)), wspec((1, 64)),
            wspec((128, 64)), wspec((1, 64)),
            wspec((16, 64)),
        ],
        out_specs=[
            pl.BlockSpec((1, _RQ, 128), lambda b, i: (b, i, 0)),
            pl.BlockSpec((1, _RQ, 128), lambda b, i: (b, i, 0)),
            pl.BlockSpec((1, _RQ, 64), lambda b, i: (b, i, 0)),
            pl.BlockSpec((1, _RQ, 64), lambda b, i: (b, i, 0)),
        ],
        out_shape=[
            jax.ShapeDtypeStruct((Bb, Nn, 128), F32),
            jax.ShapeDtypeStruct((Bb, Nn, 128), F32),
            jax.ShapeDtypeStruct((Bb, Nn, 64), F32),
            jax.ShapeDtypeStruct((Bb, Nn, 64), F32),
        ],
    )(kp, q, p16, wst, bs, w1t, b1, w2t, b2, wkt, bk, wqt, bq, wvt, bv, wut)


_GCH = 128


def _sc_gather(table, ids):
    E = ids.shape[0]
    D = table.shape[1]
    info = plsc.get_sparse_core_info()
    nw = info.num_cores * info.num_subcores
    e_per_w = E // nw
    n_ch = e_per_w // _GCH
    mesh = plsc.VectorSubcoreMesh(core_axis_name="c", subcore_axis_name="s")

    @functools.partial(
        pl.kernel, mesh=mesh,
        out_type=jax.ShapeDtypeStruct((E, D), F32),
        scratch_types=[
            pltpu.VMEM((_GCH,), jnp.int32),
            pltpu.VMEM((_GCH, D), F32),
            pltpu.SemaphoreType.DMA,
        ],
    )
    def k(table_hbm, ids_hbm, out_hbm, idx_v, rows_v, sem):
        wid = lax.axis_index("s") * info.num_cores + lax.axis_index("c")
        base = wid * e_per_w

        def body(c, _):
            off = base + c * _GCH
            pltpu.sync_copy(ids_hbm.at[pl.ds(off, _GCH)], idx_v)
            pltpu.async_copy(table_hbm.at[idx_v], rows_v, sem).wait()
            pltpu.sync_copy(rows_v, out_hbm.at[pl.ds(off, _GCH)])
            return _

        lax.fori_loop(0, n_ch, body, 0)

    return k(table, ids)


_RA = 256


def _rep(x, c):
    return jnp.broadcast_to(x[:, None, :], (_RA, KK, c)).reshape(_RA * KK, c)


def _seg(x, c):
    return jnp.sum(x.reshape(_RA, KK, c), axis=1)


def _att_body(g_ref, q_ref, v_ref, tab_ref, idn_ref,
              bp1_ref, ps_ref, pt_ref, wp2_ref, bp2_ref,
              wa1_ref, ba1_ref, as_ref, at_ref, wa2_ref, ba2_ref,
              we_ref, be_ref, wf1_ref, bf1_ref, wf2_ref, bf2_ref,
              pw_ref, h_ref, y_ref):
    g = g_ref[0]
    qu = jnp.concatenate([q_ref[0], tab_ref[0][:, 64:128]], axis=1)
    D = _rep(qu, 128) - g
    Dp = (D + bp1_ref[...]) * ps_ref[...] + pt_ref[...]
    lane = lax.broadcasted_iota(jnp.int32, (_RA * KK, 128), 1)
    Dp = jnp.where(lane >= 64, jnp.maximum(Dp, 0.0), Dp)
    pe = _dot(Dp, wp2_ref[...]) + bp2_ref[...]
    a = _dot(Dp[:, :64] + pe, wa1_ref[...]) + ba1_ref[...]
    a = jnp.maximum(a * as_ref[...] + at_ref[...], 0.0)
    logit = _dot(a, wa2_ref[...]) + ba2_ref[...]
    e2 = jnp.exp(logit - jnp.max(logit))
    val4 = jnp.concatenate(
        [_rep(v_ref[0], 64) + pe, jnp.ones((_RA * KK, 64), F32)], axis=1)
    nd = _seg(e2 * val4, 128)
    agg = nd[:, :64] / nd[:, 64:128]
    h = _dot(agg, we_ref[...]) + be_ref[...] + idn_ref[0]
    h_ref[0] = h
    fc = jnp.maximum(_dot(h, wf1_ref[...]) + bf1_ref[...], 0.0)
    fc = _dot(fc, wf2_ref[...]) + bf2_ref[...]
    y = _dot(fc, pw_ref[...])
    y_ref[0] = y.reshape(2 * _RA, 128)


def _att(g, qt, vt, tab, idn, bp1, pbs, pbt, wp2t, bp2,
         wa1t, ba1, abs_, abt, wa2t, ba2, wet, be, wf1t, bf1, wf2t, bf2,
         pw):
    wspec = lambda s: pl.BlockSpec(s, lambda b, i: (0, 0))
    return pl.pallas_call(
        _att_body,
        grid=(Bb, Nn // _RA),
        in_specs=[
            pl.BlockSpec((1, _RA * KK, 128), lambda b, i: (b, i, 0)),
            pl.BlockSpec((1, _RA, 64), lambda b, i: (b, i, 0)),
            pl.BlockSpec((1, _RA, 64), lambda b, i: (b, i, 0)),
            pl.BlockSpec((1, _RA, 128), lambda b, i: (b, i, 0)),
            pl.BlockSpec((1, _RA, 128), lambda b, i: (b, i, 0)),
            wspec((1, 128)), wspec((1, 128)), wspec((1, 128)),
            wspec((128, 64)), wspec((1, 64)),
            wspec((64, 256)), wspec((1, 256)), wspec((1, 256)), wspec((1, 256)),
            wspec((256, 128)), wspec((1, 128)),
            wspec((64, 128)), wspec((1, 128)),
            wspec((128, 64)), wspec((1, 64)),
            wspec((64, 32)), wspec((1, 32)),
            wspec((32, 256)),
        ],
        out_specs=[
            pl.BlockSpec((1, _RA, 128), lambda b, i: (b, i, 0)),
            pl.BlockSpec((1, 2 * _RA, 128), lambda b, i: (b, i, 0)),
        ],
        out_shape=[
            jax.ShapeDtypeStruct((Bb, Nn, 128), F32),
            jax.ShapeDtypeStruct((Bb, 2 * Nn, 128), F32),
        ],
    )(g, qt, vt, tab, idn, bp1, pbs, pbt, wp2t, bp2,
      wa1t, ba1, abs_, abt, wa2t, ba2, wet, be, wf1t, bf1, wf2t, bf2, pw)


_RF = 512


def _fin_body(fc_ref, h_ref, pts_ref, wsa_ref, wsb_ref, bs_ref,
              w1a_ref, w1b_ref, b1_ref, w2_ref, b2_ref,
              wd1_ref, bd1_ref, wd2_ref, bd2_ref, kc_ref, po_ref):
    fcx = fc_ref[0]
    hh = jnp.broadcast_to(h_ref[0][:, None, :], (_RF // 2, 2, 128)
                          ).reshape(_RF, 128)
    sc = _dot(fcx, wsa_ref[...]) + _dot(hh, wsb_ref[...]) + bs_ref[...]
    h = jnp.maximum(_dot(fcx, w1a_ref[...]) + _dot(hh, w1b_ref[...])
                    + b1_ref[...], 0.0)
    kc = _dot(h, w2_ref[...]) + b2_ref[...] + sc
    kc_ref[0] = jnp.transpose(kc)
    r = jnp.maximum(kc, 0.0)
    d1 = jnp.maximum(_dot(r, wd1_ref[...]) + bd1_ref[...], 0.0)
    dl = _dot(d1, wd2_ref[...]) + bd2_ref[...]
    pp = jnp.broadcast_to(pts_ref[0][:, None, :], (_RF // 2, 2, 8)
                          ).reshape(_RF, 8)
    po_ref[0] = jnp.transpose(pp + jnp.tanh(dl))


def _fin(fc_rows, H, pts8, wsat, wsbt, bs, w1at, w1bt, b1, w2t, b2,
         wd1t, bd1, wd2t, bd2):
    wspec = lambda s: pl.BlockSpec(s, lambda b, i: (0, 0))
    N2 = 2 * Nn
    return pl.pallas_call(
        _fin_body,
        grid=(Bb, N2 // _RF),
        in_specs=[
            pl.BlockSpec((1, _RF, 128), lambda b, i: (b, i, 0)),
            pl.BlockSpec((1, _RF // 2, 128), lambda b, i: (b, i, 0)),
            pl.BlockSpec((1, _RF // 2, 8), lambda b, i: (b, i, 0)),
            wspec((128, 128)), wspec((128, 128)), wspec((1, 128)),
            wspec((128, 128)), wspec((128, 128)), wspec((1, 128)),
            wspec((128, 128)), wspec((1, 128)),
            wspec((128, 64)), wspec((1, 64)),
            wspec((64, 8)), wspec((1, 8)),
        ],
        out_specs=[
            pl.BlockSpec((1, 128, _RF), lambda b, i: (b, 0, i)),
            pl.BlockSpec((1, 8, _RF), lambda b, i: (b, 0, i)),
        ],
        out_shape=[
            jax.ShapeDtypeStruct((Bb, 128, N2), F32),
            jax.ShapeDtypeStruct((Bb, 8, N2), F32),
        ],
    )(fc_rows, H, pts8, wsat, wsbt, bs, w1at, w1bt, b1, w2t, b2,
      wd1t, bd1, wd2t, bd2)


def _bn_fold(p, pre):
    s = p[pre + '_g'] / jnp.sqrt(p[pre + '_v'] + EPS)
    t = p[pre + '_b'] - p[pre + '_m'] * s
    return s[None, :], t[None, :]


def _bn_fold2(p, pre):
    s = p[pre + 'g'] / jnp.sqrt(p[pre + 'v'] + EPS)
    t = p[pre + 'b'] - p[pre + 'm'] * s
    return s[None, :], t[None, :]


def _row(v):
    return v[None, :]


def kernel(pcd_prev, feat_global, K_prev, params):
    p = params
    pts = jnp.transpose(pcd_prev, (0, 2, 1))
    pts8 = jnp.pad(pts, ((0, 0), (0, 0), (0, 5)))
    pts16 = jnp.pad(pts, ((0, 0), (0, 0), (0, 13)))

    idx, idx2 = _knn(pts8)

    x = _mlp1(pts8.reshape(Bb * Nn, 8),
              jnp.pad(p['mlp1_W1'].T, ((0, 5), (0, 0))), _row(p['mlp1_b1']),
              p['mlp1_W2'].T, _row(p['mlp1_b2'])).reshape(Bb, Nn, 128)

    s1, t1 = _bn_fold(p, 'bn1')
    h1 = _sage(x, idx, p['sage1_Wl'].T, _row(p['sage1_bl']),
               p['sage1_Wr'].T, s1, t1, residual=False)
    s2, t2 = _bn_fold(p, 'bn2')
    h2 = _sage(h1, idx, p['sage2_Wl'].T, _row(p['sage2_bl']),
               p['sage2_Wr'].T, s2, t2, residual=True)

    fm = _fmax(h2)
    fg = jnp.transpose(feat_global, (0, 2, 1))
    W1 = p['mlp2_W1']
    Q = _qkern(h2, fm, fg,
               W1[:, :128].T, W1[:, 128:256].T, W1[:, 256:].T,
               _row(p['mlp2_b1']), p['mlp2_W2'].T, _row(p['mlp2_b2']))

    value, tab, query_t, value_t = _kqv(
        K_prev, Q, pts16,
        p['st_mlpv_Ws'].T, _row(p['st_mlpv_bs']),
        p['st_mlpv_W1'].T, _row(p['st_mlpv_b1']),
        p['st_mlpv_W2'].T, _row(p['st_mlpv_b2']),
        p['st_Wk'].T, _row(p['st_bk']),
        p['st_Wq'].T, _row(p['st_bq']),
        p['st_Wv'].T, _row(p['st_bv']),
        jnp.pad(p['st_pos_W1'].T, ((0, 13), (0, 0))))

    flat_ids = (idx2 + (jnp.arange(Bb, dtype=jnp.int32) * Nn)[:, None, None]
                ).reshape(-1)
    g = _sc_gather(tab.reshape(Bb * Nn, 128), flat_ids
                   ).reshape(Bb, Nn * KK, 128)

    pbs, pbt = _bn_fold2(p, 'st_pos_bn')
    abs_, abt = _bn_fold2(p, 'st_att_bn')
    z64 = jnp.zeros((1, 64), F32)
    o64 = jnp.ones((1, 64), F32)
    wa2d = p['st_att_W2'].T
    H, fc_rows = _att(
        g, query_t, value_t, tab, value,
        jnp.concatenate([z64, _row(p['st_pos_b1'])], axis=1),
        jnp.concatenate([o64, pbs], axis=1),
        jnp.concatenate([z64, pbt], axis=1),
        jnp.concatenate([jnp.zeros((64, 64), F32), p['st_pos_W2'].T], axis=0),
        _row(p['st_pos_b2']),
        p['st_att_W1'].T, _row(p['st_att_b1']), abs_, abt,
        jnp.concatenate([wa2d, wa2d], axis=1),
        jnp.concatenate([_row(p['st_att_b2']), _row(p['st_att_b2'])], axis=1),
        p['st_We'].T, _row(p['st_be']),
        p['mlpps_W1'].T, _row(p['mlpps_b1']),
        p['mlpps_W2'].T, _row(p['mlpps_b2']),
        jnp.transpose(p['ps_W'], (0, 2, 1)).reshape(32, 256))

    Wdfs = p['mlpdf_Ws'].T
    Wdf1 = p['mlpdf_W1'].T
    kc, po = _fin(fc_rows, H, pts8,
                  Wdfs[:128], Wdfs[128:], _row(p['mlpdf_bs']),
                  Wdf1[:128], Wdf1[128:], _row(p['mlpdf_b1']),
                  p['mlpdf_W2'].T, _row(p['mlpdf_b2']),
                  p['mlpd_W1'].T, _row(p['mlpd_b1']),
                  jnp.pad(p['mlpd_W2'].T, ((0, 0), (0, 5))),
                  jnp.pad(_row(p['mlpd_b2']), ((0, 0), (0, 5))))

    return po[:, :3, :], kc


# speedup vs baseline: 1.3567x; 1.0936x over previous
import functools

import jax
import jax.numpy as jnp
from jax import lax
from jax.experimental import pallas as pl
from jax.experimental.pallas import tpu as pltpu
from jax.experimental.pallas import tpu_sc as plsc

Bb, Nn, DFf, KK = 4, 2048, 512, 16
EPS = 1e-5
F32 = jnp.float32


def _hi(x):
    return x


def _dotT(a, b):
    return lax.dot_general(a, b, (((1,), (1,)), ((), ())),
                           preferred_element_type=F32)


def _dot(a, b):
    return lax.dot_general(a, b, (((1,), (0,)), ((), ())),
                           preferred_element_type=F32)


_RBK = 256


def _knn_body(ptsb_ref, ptsf_ref, ones_ref, idx_ref, idx2_ref):
    pb = ptsb_ref[0]
    pf = ptsf_ref[0]
    sq_b = jnp.sum(pb * pb, axis=1, keepdims=True)
    sq_f = _dotT(ones_ref[...], pf * pf)
    g = _dotT(pb, pf)
    d = sq_b + sq_f - 2.0 * g
    col = lax.broadcasted_iota(jnp.int32, (_RBK, Nn), 1)
    big = jnp.float32(1e30)
    i = pl.program_id(1)
    n_self = i * _RBK + lax.broadcasted_iota(jnp.int32, (_RBK, 1), 0)
    dd = jnp.where(col == n_self, big, d)
    colf = col.astype(F32)
    ams = []
    for _ in range(KK):
        dmin = jnp.min(dd, axis=1, keepdims=True)
        eqm = dd == dmin
        am = jnp.min(jnp.where(eqm, colf, jnp.float32(Nn)), axis=1,
                     keepdims=True)
        ams.append(am)
        dd = jnp.where(eqm, big, dd)
    v = jnp.concatenate(ams, axis=1).astype(jnp.int32)
    idx_ref[0] = v
    idx2_ref[0] = jnp.concatenate([n_self, v[:, :KK - 1]], axis=1)


def _knn(pts8):
    grid = (Bb, Nn // _RBK)
    out = pl.pallas_call(
        _knn_body,
        grid=grid,
        in_specs=[
            pl.BlockSpec((1, _RBK, 8), lambda b, i: (b, i, 0)),
            pl.BlockSpec((1, Nn, 8), lambda b, i: (b, 0, 0)),
            pl.BlockSpec((1, 8), lambda b, i: (0, 0)),
        ],
        out_specs=[
            pl.BlockSpec((1, _RBK, KK), lambda b, i: (b, i, 0)),
            pl.BlockSpec((1, _RBK, KK), lambda b, i: (b, i, 0)),
        ],
        out_shape=[
            jax.ShapeDtypeStruct((Bb, Nn, KK), jnp.int32),
            jax.ShapeDtypeStruct((Bb, Nn, KK), jnp.int32),
        ],
    )(pts8, pts8, jnp.ones((1, 8), F32))
    return out


def _mlp1_body(x_ref, w1_ref, b1_ref, w2_ref, b2_ref, o_ref):
    h = jnp.maximum(_dot(x_ref[...], w1_ref[...]) + b1_ref[...], 0.0)
    o_ref[...] = _dot(h, w2_ref[...]) + b2_ref[...]


def _mlp1(xrows, w1t, b1, w2t, b2):
    R = xrows.shape[0]
    RB = 1024
    return pl.pallas_call(
        _mlp1_body,
        grid=(R // RB,),
        in_specs=[
            pl.BlockSpec((RB, 8), lambda i: (i, 0)),
            pl.BlockSpec(w1t.shape, lambda i: (0, 0)),
            pl.BlockSpec(b1.shape, lambda i: (0, 0)),
            pl.BlockSpec(w2t.shape, lambda i: (0, 0)),
            pl.BlockSpec(b2.shape, lambda i: (0, 0)),
        ],
        out_specs=pl.BlockSpec((RB, 128), lambda i: (i, 0)),
        out_shape=jax.ShapeDtypeStruct((R, 128), F32),
    )(xrows, w1t, b1, w2t, b2)


_RBS = 256


def _sage_body(idx_ref, xf_ref, xb_ref, wl_ref, wr_ref, cb_ref, sc_ref,
               sh_ref, o_ref, *, residual):
    idx = idx_ref[0]
    xf = xf_ref[0]
    xb = xb_ref[0]
    col = lax.broadcasted_iota(jnp.int32, (_RBS, Nn), 1)
    acc = jnp.zeros((_RBS, Nn), F32)
    for k in range(KK):
        acc = acc + (col == idx[:, k:k + 1]).astype(F32)
    mean = _dot(acc, xf) * (1.0 / KK)
    y = _dot(mean, wl_ref[...]) + cb_ref[...] + _dot(xb, wr_ref[...])
    y = jnp.maximum(y * sc_ref[...] + sh_ref[...], 0.0)
    if residual:
        y = y + xb
    o_ref[0] = y


def _sage(x, idx, wlt, bl, wrt, bn_s, bn_t, residual):
    body = functools.partial(_sage_body, residual=residual)
    return pl.pallas_call(
        body,
        grid=(Bb, Nn // _RBS),
        in_specs=[
            pl.BlockSpec((1, _RBS, KK), lambda b, i: (b, i, 0)),
            pl.BlockSpec((1, Nn, 128), lambda b, i: (b, 0, 0)),
            pl.BlockSpec((1, _RBS, 128), lambda b, i: (b, i, 0)),
            pl.BlockSpec((128, 128), lambda b, i: (0, 0)),
            pl.BlockSpec((128, 128), lambda b, i: (0, 0)),
            pl.BlockSpec((1, 128), lambda b, i: (0, 0)),
            pl.BlockSpec((1, 128), lambda b, i: (0, 0)),
            pl.BlockSpec((1, 128), lambda b, i: (0, 0)),
        ],
        out_specs=pl.BlockSpec((1, _RBS, 128), lambda b, i: (b, i, 0)),
        out_shape=jax.ShapeDtypeStruct((Bb, Nn, 128), F32),
    )(idx, x, x, wlt, wrt, bl, bn_s, bn_t)


def _fmax_body(x_ref, o_ref):
    o_ref[0] = jnp.max(x_ref[0], axis=0, keepdims=True)


def _fmax(h2):
    return pl.pallas_call(
        _fmax_body,
        grid=(Bb,),
        in_specs=[pl.BlockSpec((1, Nn, 128), lambda b: (b, 0, 0))],
        out_specs=pl.BlockSpec((1, 1, 128), lambda b: (b, 0, 0)),
        out_shape=jax.ShapeDtypeStruct((Bb, 1, 128), F32),
    )(h2)


_RQ = 512


def _q_body(x_ref, fm_ref, fg_ref, w1a_ref, w1b_ref, w1c_ref, b1_ref,
            w2_ref, b2_ref, o_ref):
    c = (_dot(fm_ref[0], w1b_ref[...]) + _dot(fg_ref[0], w1c_ref[...])
         + b1_ref[...])
    h = jnp.maximum(_dot(x_ref[0], w1a_ref[...]) + c, 0.0)
    o_ref[0] = _dot(h, w2_ref[...]) + b2_ref[...]


def _qkern(h2, fm, fg, w1at, w1bt, w1ct, b1, w2t, b2):
    return pl.pallas_call(
        _q_body,
        grid=(Bb, Nn // _RQ),
        in_specs=[
            pl.BlockSpec((1, _RQ, 128), lambda b, i: (b, i, 0)),
            pl.BlockSpec((1, 1, 128), lambda b, i: (b, 0, 0)),
            pl.BlockSpec((1, 1, DFf), lambda b, i: (b, 0, 0)),
            pl.BlockSpec((128, 256), lambda b, i: (0, 0)),
            pl.BlockSpec((128, 256), lambda b, i: (0, 0)),
            pl.BlockSpec((DFf, 256), lambda b, i: (0, 0)),
            pl.BlockSpec((1, 256), lambda b, i: (0, 0)),
            pl.BlockSpec((256, 128), lambda b, i: (0, 0)),
            pl.BlockSpec((1, 128), lambda b, i: (0, 0)),
        ],
        out_specs=pl.BlockSpec((1, _RQ, 128), lambda b, i: (b, i, 0)),
        out_shape=jax.ShapeDtypeStruct((Bb, Nn, 128), F32),
    )(h2, fm, fg, w1at, w1bt, w1ct, b1, w2t, b2)


def _tab_body(kp_ref, p16_ref, wk_ref, bk_ref, wu_ref, tab_ref):
    kp = jnp.transpose(kp_ref[0])
    key = _dot(kp, wk_ref[...]) + bk_ref[...]
    u = _dot(p16_ref[0], wu_ref[...])
    tab_ref[0] = jnp.concatenate([key, u], axis=1)


def _tab_kern(K_prev, p16, wkt, bk, wut):
    wspec = lambda s: pl.BlockSpec(s, lambda b, i: (0, 0))
    return pl.pallas_call(
        _tab_body,
        grid=(Bb, Nn // _RQ),
        in_specs=[
            pl.BlockSpec((1, 128, _RQ), lambda b, i: (b, 0, i)),
            pl.BlockSpec((1, _RQ, 16), lambda b, i: (b, i, 0)),
            wspec((128, 64)), wspec((1, 64)), wspec((16, 64)),
        ],
        out_specs=pl.BlockSpec((1, _RQ, 128), lambda b, i: (b, i, 0)),
        out_shape=jax.ShapeDtypeStruct((Bb, Nn, 128), F32),
    )(K_prev, p16, wkt, bk, wut)


def _kqv_body(kp_ref, q_ref, wst_ref, bs_ref, w1_ref, b1_ref,
              w2_ref, b2_ref, wq_ref, bq_ref, wv_ref,
              bv_ref, val_ref, qry_ref, vt_ref):
    kp = jnp.transpose(kp_ref[0])
    q = q_ref[0]
    cat = jnp.concatenate([kp, q], axis=1)
    sc = _dot(cat, wst_ref[...]) + bs_ref[...]
    h = jnp.maximum(_dot(cat, w1_ref[...]) + b1_ref[...], 0.0)
    val = _dot(h, w2_ref[...]) + b2_ref[...] + sc
    val_ref[0] = val
    qry_ref[0] = _dot(q, wq_ref[...]) + bq_ref[...]
    vt_ref[0] = _dot(val, wv_ref[...]) + bv_ref[...]


def _kqv(kp, q, wst, bs, w1t, b1, w2t, b2, wqt, bq, wvt, bv):
    wspec = lambda s: pl.BlockSpec(s, lambda b, i: (0, 0))
    return pl.pallas_call(
        _kqv_body,
        grid=(Bb, Nn // _RQ),
        in_specs=[
            pl.BlockSpec((1, 128, _RQ), lambda b, i: (b, 0, i)),
            pl.BlockSpec((1, _RQ, 128), lambda b, i: (b, i, 0)),
            wspec((256, 128)), wspec((1, 128)),
            wspec((256, 128)), wspec((1, 128)),
            wspec((128, 128)), wspec((1, 128)),
            wspec((128, 64)), wspec((1, 64)),
            wspec((128, 64)), wspec((1, 64)),
        ],
        out_specs=[
            pl.BlockSpec((1, _RQ, 128), lambda b, i: (b, i, 0)),
            pl.BlockSpec((1, _RQ, 64), lambda b, i: (b, i, 0)),
            pl.BlockSpec((1, _RQ, 64), lambda b, i: (b, i, 0)),
        ],
        out_shape=[
            jax.ShapeDtypeStruct((Bb, Nn, 128), F32),
            jax.ShapeDtypeStruct((Bb, Nn, 64), F32),
            jax.ShapeDtypeStruct((Bb, Nn, 64), F32),
        ],
    )(kp, q, wst, bs, w1t, b1, w2t, b2, wqt, bq, wvt, bv)


_GCH = 128


def _sc_gather(table, ids):
    E = ids.shape[0]
    D = table.shape[1]
    info = plsc.get_sparse_core_info()
    nw = info.num_cores * info.num_subcores
    e_per_w = E // nw
    n_ch = e_per_w // _GCH
    mesh = plsc.VectorSubcoreMesh(core_axis_name="c", subcore_axis_name="s")

    @functools.partial(
        pl.kernel, mesh=mesh,
        out_type=jax.ShapeDtypeStruct((E, D), F32),
        scratch_types=[
            pltpu.VMEM((_GCH,), jnp.int32),
            pltpu.VMEM((_GCH, D), F32),
            pltpu.SemaphoreType.DMA,
        ],
    )
    def k(table_hbm, ids_hbm, out_hbm, idx_v, rows_v, sem):
        wid = lax.axis_index("s") * info.num_cores + lax.axis_index("c")
        base = wid * e_per_w

        def body(c, _):
            off = base + c * _GCH
            pltpu.sync_copy(ids_hbm.at[pl.ds(off, _GCH)], idx_v)
            pltpu.async_copy(table_hbm.at[idx_v], rows_v, sem).wait()
            pltpu.sync_copy(rows_v, out_hbm.at[pl.ds(off, _GCH)])
            return _

        lax.fori_loop(0, n_ch, body, 0)

    return k(table, ids)


_RA = 256


def _rep(x, c):
    return jnp.broadcast_to(x[:, None, :], (_RA, KK, c)).reshape(_RA * KK, c)


def _seg(x, c):
    return jnp.sum(x.reshape(_RA, KK, c), axis=1)


def _att_body(g_ref, q_ref, v_ref, tab_ref, idn_ref,
              bp1_ref, ps_ref, pt_ref, wp2_ref, bp2_ref,
              wa1_ref, ba1_ref, as_ref, at_ref, wa2_ref, ba2_ref,
              we_ref, be_ref, wf1_ref, bf1_ref, wf2_ref, bf2_ref,
              pw_ref, h_ref, y_ref):
    g = g_ref[0]
    qu = jnp.concatenate([q_ref[0], tab_ref[0][:, 64:128]], axis=1)
    D = _rep(qu, 128) - g
    Dp = (D + bp1_ref[...]) * ps_ref[...] + pt_ref[...]
    lane = lax.broadcasted_iota(jnp.int32, (_RA * KK, 128), 1)
    Dp = jnp.where(lane >= 64, jnp.maximum(Dp, 0.0), Dp)
    pe = _dot(Dp, wp2_ref[...]) + bp2_ref[...]
    a = _dot(Dp[:, :64] + pe, wa1_ref[...]) + ba1_ref[...]
    a = jnp.maximum(a * as_ref[...] + at_ref[...], 0.0)
    logit = _dot(a, wa2_ref[...]) + ba2_ref[...]
    e2 = jnp.exp(logit - jnp.max(logit))
    val4 = jnp.concatenate(
        [_rep(v_ref[0], 64) + pe, jnp.ones((_RA * KK, 64), F32)], axis=1)
    nd = _seg(e2 * val4, 128)
    agg = nd[:, :64] / nd[:, 64:128]
    h = _dot(agg, we_ref[...]) + be_ref[...] + idn_ref[0]
    h_ref[0] = h
    fc = jnp.maximum(_dot(h, wf1_ref[...]) + bf1_ref[...], 0.0)
    fc = _dot(fc, wf2_ref[...]) + bf2_ref[...]
    y = _dot(fc, pw_ref[...])
    y_ref[0] = y.reshape(2 * _RA, 128)


def _att(g, qt, vt, tab, idn, bp1, pbs, pbt, wp2t, bp2,
         wa1t, ba1, abs_, abt, wa2t, ba2, wet, be, wf1t, bf1, wf2t, bf2,
         pw):
    wspec = lambda s: pl.BlockSpec(s, lambda b, i: (0, 0))
    return pl.pallas_call(
        _att_body,
        grid=(Bb, Nn // _RA),
        in_specs=[
            pl.BlockSpec((1, _RA * KK, 128), lambda b, i: (b, i, 0)),
            pl.BlockSpec((1, _RA, 64), lambda b, i: (b, i, 0)),
            pl.BlockSpec((1, _RA, 64), lambda b, i: (b, i, 0)),
            pl.BlockSpec((1, _RA, 128), lambda b, i: (b, i, 0)),
            pl.BlockSpec((1, _RA, 128), lambda b, i: (b, i, 0)),
            wspec((1, 128)), wspec((1, 128)), wspec((1, 128)),
            wspec((128, 64)), wspec((1, 64)),
            wspec((64, 256)), wspec((1, 256)), wspec((1, 256)), wspec((1, 256)),
            wspec((256, 128)), wspec((1, 128)),
            wspec((64, 128)), wspec((1, 128)),
            wspec((128, 64)), wspec((1, 64)),
            wspec((64, 32)), wspec((1, 32)),
            wspec((32, 256)),
        ],
        out_specs=[
            pl.BlockSpec((1, _RA, 128), lambda b, i: (b, i, 0)),
            pl.BlockSpec((1, 2 * _RA, 128), lambda b, i: (b, i, 0)),
        ],
        out_shape=[
            jax.ShapeDtypeStruct((Bb, Nn, 128), F32),
            jax.ShapeDtypeStruct((Bb, 2 * Nn, 128), F32),
        ],
    )(g, qt, vt, tab, idn, bp1, pbs, pbt, wp2t, bp2,
      wa1t, ba1, abs_, abt, wa2t, ba2, wet, be, wf1t, bf1, wf2t, bf2, pw)


_RF = 512


def _fin_body(fc_ref, h_ref, pts_ref, wsa_ref, wsb_ref, bs_ref,
              w1a_ref, w1b_ref, b1_ref, w2_ref, b2_ref,
              wd1_ref, bd1_ref, wd2_ref, bd2_ref, kc_ref, po_ref):
    fcx = fc_ref[0]
    hh = jnp.broadcast_to(h_ref[0][:, None, :], (_RF // 2, 2, 128)
                          ).reshape(_RF, 128)
    sc = _dot(fcx, wsa_ref[...]) + _dot(hh, wsb_ref[...]) + bs_ref[...]
    h = jnp.maximum(_dot(fcx, w1a_ref[...]) + _dot(hh, w1b_ref[...])
                    + b1_ref[...], 0.0)
    kc = _dot(h, w2_ref[...]) + b2_ref[...] + sc
    kc_ref[0] = jnp.transpose(kc)
    r = jnp.maximum(kc, 0.0)
    d1 = jnp.maximum(_dot(r, wd1_ref[...]) + bd1_ref[...], 0.0)
    dl = _dot(d1, wd2_ref[...]) + bd2_ref[...]
    pp = jnp.broadcast_to(pts_ref[0][:, None, :], (_RF // 2, 2, 8)
                          ).reshape(_RF, 8)
    po_ref[0] = jnp.transpose(pp + jnp.tanh(dl))


def _fin(fc_rows, H, pts8, wsat, wsbt, bs, w1at, w1bt, b1, w2t, b2,
         wd1t, bd1, wd2t, bd2):
    wspec = lambda s: pl.BlockSpec(s, lambda b, i: (0, 0))
    N2 = 2 * Nn
    return pl.pallas_call(
        _fin_body,
        grid=(Bb, N2 // _RF),
        in_specs=[
            pl.BlockSpec((1, _RF, 128), lambda b, i: (b, i, 0)),
            pl.BlockSpec((1, _RF // 2, 128), lambda b, i: (b, i, 0)),
            pl.BlockSpec((1, _RF // 2, 8), lambda b, i: (b, i, 0)),
            wspec((128, 128)), wspec((128, 128)), wspec((1, 128)),
            wspec((128, 128)), wspec((128, 128)), wspec((1, 128)),
            wspec((128, 128)), wspec((1, 128)),
            wspec((128, 64)), wspec((1, 64)),
            wspec((64, 8)), wspec((1, 8)),
        ],
        out_specs=[
            pl.BlockSpec((1, 128, _RF), lambda b, i: (b, 0, i)),
            pl.BlockSpec((1, 8, _RF), lambda b, i: (b, 0, i)),
        ],
        out_shape=[
            jax.ShapeDtypeStruct((Bb, 128, N2), F32),
            jax.ShapeDtypeStruct((Bb, 8, N2), F32),
        ],
    )(fc_rows, H, pts8, wsat, wsbt, bs, w1at, w1bt, b1, w2t, b2,
      wd1t, bd1, wd2t, bd2)


def _bn_fold(p, pre):
    s = p[pre + '_g'] / jnp.sqrt(p[pre + '_v'] + EPS)
    t = p[pre + '_b'] - p[pre + '_m'] * s
    return s[None, :], t[None, :]


def _bn_fold2(p, pre):
    s = p[pre + 'g'] / jnp.sqrt(p[pre + 'v'] + EPS)
    t = p[pre + 'b'] - p[pre + 'm'] * s
    return s[None, :], t[None, :]


def _row(v):
    return v[None, :]


def kernel(pcd_prev, feat_global, K_prev, params):
    p = params
    pts = jnp.transpose(pcd_prev, (0, 2, 1))
    pts8 = jnp.pad(pts, ((0, 0), (0, 0), (0, 5)))
    pts16 = jnp.pad(pts, ((0, 0), (0, 0), (0, 13)))

    idx, idx2 = _knn(pts8)

    tab = _tab_kern(K_prev, pts16, p['st_Wk'].T, _row(p['st_bk']),
                    jnp.pad(p['st_pos_W1'].T, ((0, 13), (0, 0))))
    flat_ids = (idx2 + (jnp.arange(Bb, dtype=jnp.int32) * Nn)[:, None, None]
                ).reshape(-1)
    g = _sc_gather(tab.reshape(Bb * Nn, 128), flat_ids
                   ).reshape(Bb, Nn * KK, 128)

    x = _mlp1(pts8.reshape(Bb * Nn, 8),
              jnp.pad(p['mlp1_W1'].T, ((0, 5), (0, 0))), _row(p['mlp1_b1']),
              p['mlp1_W2'].T, _row(p['mlp1_b2'])).reshape(Bb, Nn, 128)

    s1, t1 = _bn_fold(p, 'bn1')
    h1 = _sage(x, idx, p['sage1_Wl'].T, _row(p['sage1_bl']),
               p['sage1_Wr'].T, s1, t1, residual=False)
    s2, t2 = _bn_fold(p, 'bn2')
    h2 = _sage(h1, idx, p['sage2_Wl'].T, _row(p['sage2_bl']),
               p['sage2_Wr'].T, s2, t2, residual=True)

    fm = _fmax(h2)
    fg = jnp.transpose(feat_global, (0, 2, 1))
    W1 = p['mlp2_W1']
    Q = _qkern(h2, fm, fg,
               W1[:, :128].T, W1[:, 128:256].T, W1[:, 256:].T,
               _row(p['mlp2_b1']), p['mlp2_W2'].T, _row(p['mlp2_b2']))

    value, query_t, value_t = _kqv(
        K_prev, Q,
        p['st_mlpv_Ws'].T, _row(p['st_mlpv_bs']),
        p['st_mlpv_W1'].T, _row(p['st_mlpv_b1']),
        p['st_mlpv_W2'].T, _row(p['st_mlpv_b2']),
        p['st_Wq'].T, _row(p['st_bq']),
        p['st_Wv'].T, _row(p['st_bv']))

    pbs, pbt = _bn_fold2(p, 'st_pos_bn')
    abs_, abt = _bn_fold2(p, 'st_att_bn')
    z64 = jnp.zeros((1, 64), F32)
    o64 = jnp.ones((1, 64), F32)
    wa2d = p['st_att_W2'].T
    H, fc_rows = _att(
        g, query_t, value_t, tab, value,
        jnp.concatenate([z64, _row(p['st_pos_b1'])], axis=1),
        jnp.concatenate([o64, pbs], axis=1),
        jnp.concatenate([z64, pbt], axis=1),
        jnp.concatenate([jnp.zeros((64, 64), F32), p['st_pos_W2'].T], axis=0),
        _row(p['st_pos_b2']),
        p['st_att_W1'].T, _row(p['st_att_b1']), abs_, abt,
        jnp.concatenate([wa2d, wa2d], axis=1),
        jnp.concatenate([_row(p['st_att_b2']), _row(p['st_att_b2'])], axis=1),
        p['st_We'].T, _row(p['st_be']),
        p['mlpps_W1'].T, _row(p['mlpps_b1']),
        p['mlpps_W2'].T, _row(p['mlpps_b2']),
        jnp.transpose(p['ps_W'], (0, 2, 1)).reshape(32, 256))

    Wdfs = p['mlpdf_Ws'].T
    Wdf1 = p['mlpdf_W1'].T
    kc, po = _fin(fc_rows, H, pts8,
                  Wdfs[:128], Wdfs[128:], _row(p['mlpdf_bs']),
                  Wdf1[:128], Wdf1[128:], _row(p['mlpdf_b1']),
                  p['mlpdf_W2'].T, _row(p['mlpdf_b2']),
                  p['mlpd_W1'].T, _row(p['mlpd_b1']),
                  jnp.pad(p['mlpd_W2'].T, ((0, 0), (0, 5))),
                  jnp.pad(_row(p['mlpd_b2']), ((0, 0), (0, 5))))

    return po[:, :3, :], kc


# speedup vs baseline: 1.5321x; 1.1294x over previous
import functools

import jax
import jax.numpy as jnp
from jax import lax
from jax.experimental import pallas as pl
from jax.experimental.pallas import tpu as pltpu
from jax.experimental.pallas import tpu_sc as plsc

Bb, Nn, DFf, KK = 4, 2048, 512, 16
EPS = 1e-5
F32 = jnp.float32


def _hi(x):
    return x


def _dotT(a, b):
    return lax.dot_general(a, b, (((1,), (1,)), ((), ())),
                           preferred_element_type=F32)


def _dot(a, b):
    return lax.dot_general(a, b, (((1,), (0,)), ((), ())),
                           preferred_element_type=F32)


_RBK = 256


def _knn_body(ptsb_ref, ptsf_ref, ones_ref, idx_ref, idx2_ref):
    pb = ptsb_ref[0]
    pf = ptsf_ref[0]
    sq_b = jnp.sum(pb * pb, axis=1, keepdims=True)
    sq_f = _dotT(ones_ref[...], pf * pf)
    g = _dotT(pb, pf)
    d = sq_b + sq_f - 2.0 * g
    col = lax.broadcasted_iota(jnp.int32, (_RBK, Nn), 1)
    big = jnp.float32(1e30)
    i = pl.program_id(1)
    n_self = i * _RBK + lax.broadcasted_iota(jnp.int32, (_RBK, 1), 0)
    dd = jnp.where(col == n_self, big, d)
    colf = col.astype(F32)
    ams = []
    for _ in range(KK):
        dmin = jnp.min(dd, axis=1, keepdims=True)
        eqm = dd == dmin
        am = jnp.min(jnp.where(eqm, colf, jnp.float32(Nn)), axis=1,
                     keepdims=True)
        ams.append(am)
        dd = jnp.where(eqm, big, dd)
    v = jnp.concatenate(ams, axis=1).astype(jnp.int32)
    idx_ref[0] = v
    idx2_ref[0] = jnp.concatenate([n_self, v[:, :KK - 1]], axis=1)


def _knn(pts8):
    grid = (Bb, Nn // _RBK)
    out = pl.pallas_call(
        _knn_body,
        grid=grid,
        in_specs=[
            pl.BlockSpec((1, _RBK, 8), lambda b, i: (b, i, 0)),
            pl.BlockSpec((1, Nn, 8), lambda b, i: (b, 0, 0)),
            pl.BlockSpec((1, 8), lambda b, i: (0, 0)),
        ],
        out_specs=[
            pl.BlockSpec((1, _RBK, KK), lambda b, i: (b, i, 0)),
            pl.BlockSpec((1, _RBK, KK), lambda b, i: (b, i, 0)),
        ],
        out_shape=[
            jax.ShapeDtypeStruct((Bb, Nn, KK), jnp.int32),
            jax.ShapeDtypeStruct((Bb, Nn, KK), jnp.int32),
        ],
    )(pts8, pts8, jnp.ones((1, 8), F32))
    return out


def _mlp1_body(x_ref, w1_ref, b1_ref, w2_ref, b2_ref, o_ref):
    h = jnp.maximum(_dot(x_ref[...], w1_ref[...]) + b1_ref[...], 0.0)
    o_ref[...] = _dot(h, w2_ref[...]) + b2_ref[...]


def _mlp1(xrows, w1t, b1, w2t, b2):
    R = xrows.shape[0]
    RB = 1024
    return pl.pallas_call(
        _mlp1_body,
        grid=(R // RB,),
        in_specs=[
            pl.BlockSpec((RB, 8), lambda i: (i, 0)),
            pl.BlockSpec(w1t.shape, lambda i: (0, 0)),
            pl.BlockSpec(b1.shape, lambda i: (0, 0)),
            pl.BlockSpec(w2t.shape, lambda i: (0, 0)),
            pl.BlockSpec(b2.shape, lambda i: (0, 0)),
        ],
        out_specs=pl.BlockSpec((RB, 128), lambda i: (i, 0)),
        out_shape=jax.ShapeDtypeStruct((R, 128), F32),
    )(xrows, w1t, b1, w2t, b2)


_RBS = 256


def _sage1_body(idx_ref, xf_ref, xb_ref, wl_ref, wr_ref, cb_ref, sc_ref,
                sh_ref, o_ref, a_ref):
    idx = idx_ref[0]
    xf = xf_ref[0]
    xb = xb_ref[0]
    col = lax.broadcasted_iota(jnp.int32, (_RBS, Nn), 1)
    acc = jnp.zeros((_RBS, Nn), F32)
    for k in range(KK):
        acc = acc + (col == idx[:, k:k + 1]).astype(F32)
    a_ref[0] = acc.astype(jnp.bfloat16)
    mean = _dot(acc, xf) * (1.0 / KK)
    y = _dot(mean, wl_ref[...]) + cb_ref[...] + _dot(xb, wr_ref[...])
    y = jnp.maximum(y * sc_ref[...] + sh_ref[...], 0.0)
    o_ref[0] = y


def _sage1(x, idx, wlt, bl, wrt, bn_s, bn_t):
    return pl.pallas_call(
        _sage1_body,
        grid=(Bb, Nn // _RBS),
        in_specs=[
            pl.BlockSpec((1, _RBS, KK), lambda b, i: (b, i, 0)),
            pl.BlockSpec((1, Nn, 128), lambda b, i: (b, 0, 0)),
            pl.BlockSpec((1, _RBS, 128), lambda b, i: (b, i, 0)),
            pl.BlockSpec((128, 128), lambda b, i: (0, 0)),
            pl.BlockSpec((128, 128), lambda b, i: (0, 0)),
            pl.BlockSpec((1, 128), lambda b, i: (0, 0)),
            pl.BlockSpec((1, 128), lambda b, i: (0, 0)),
            pl.BlockSpec((1, 128), lambda b, i: (0, 0)),
        ],
        out_specs=[
            pl.BlockSpec((1, _RBS, 128), lambda b, i: (b, i, 0)),
            pl.BlockSpec((1, _RBS, Nn), lambda b, i: (b, i, 0)),
        ],
        out_shape=[
            jax.ShapeDtypeStruct((Bb, Nn, 128), F32),
            jax.ShapeDtypeStruct((Bb, Nn, Nn), jnp.bfloat16),
        ],
    )(idx, x, x, wlt, wrt, bl, bn_s, bn_t)


def _sage2_body(a_ref, xf_ref, xb_ref, wl_ref, wr_ref, cb_ref, sc_ref,
                sh_ref, o_ref):
    acc = a_ref[0].astype(F32)
    mean = _dot(acc, xf_ref[0]) * (1.0 / KK)
    xb = xb_ref[0]
    y = _dot(mean, wl_ref[...]) + cb_ref[...] + _dot(xb, wr_ref[...])
    y = jnp.maximum(y * sc_ref[...] + sh_ref[...], 0.0)
    o_ref[0] = y + xb


def _sage2(x, A, wlt, bl, wrt, bn_s, bn_t):
    return pl.pallas_call(
        _sage2_body,
        grid=(Bb, Nn // _RBS),
        in_specs=[
            pl.BlockSpec((1, _RBS, Nn), lambda b, i: (b, i, 0)),
            pl.BlockSpec((1, Nn, 128), lambda b, i: (b, 0, 0)),
            pl.BlockSpec((1, _RBS, 128), lambda b, i: (b, i, 0)),
            pl.BlockSpec((128, 128), lambda b, i: (0, 0)),
            pl.BlockSpec((128, 128), lambda b, i: (0, 0)),
            pl.BlockSpec((1, 128), lambda b, i: (0, 0)),
            pl.BlockSpec((1, 128), lambda b, i: (0, 0)),
            pl.BlockSpec((1, 128), lambda b, i: (0, 0)),
        ],
        out_specs=pl.BlockSpec((1, _RBS, 128), lambda b, i: (b, i, 0)),
        out_shape=jax.ShapeDtypeStruct((Bb, Nn, 128), F32),
    )(A, x, x, wlt, wrt, bl, bn_s, bn_t)


def _fmax_body(x_ref, o_ref):
    o_ref[0] = jnp.max(x_ref[0], axis=0, keepdims=True)


def _fmax(h2):
    return pl.pallas_call(
        _fmax_body,
        grid=(Bb,),
        in_specs=[pl.BlockSpec((1, Nn, 128), lambda b: (b, 0, 0))],
        out_specs=pl.BlockSpec((1, 1, 128), lambda b: (b, 0, 0)),
        out_shape=jax.ShapeDtypeStruct((Bb, 1, 128), F32),
    )(h2)


_RQ = 512


def _q_body(x_ref, fm_ref, fg_ref, w1a_ref, w1b_ref, w1c_ref, b1_ref,
            w2_ref, b2_ref, o_ref):
    c = (_dot(fm_ref[0], w1b_ref[...]) + _dot(fg_ref[0], w1c_ref[...])
         + b1_ref[...])
    h = jnp.maximum(_dot(x_ref[0], w1a_ref[...]) + c, 0.0)
    o_ref[0] = _dot(h, w2_ref[...]) + b2_ref[...]


def _qkern(h2, fm, fg, w1at, w1bt, w1ct, b1, w2t, b2):
    return pl.pallas_call(
        _q_body,
        grid=(Bb, Nn // _RQ),
        in_specs=[
            pl.BlockSpec((1, _RQ, 128), lambda b, i: (b, i, 0)),
            pl.BlockSpec((1, 1, 128), lambda b, i: (b, 0, 0)),
            pl.BlockSpec((1, 1, DFf), lambda b, i: (b, 0, 0)),
            pl.BlockSpec((128, 256), lambda b, i: (0, 0)),
            pl.BlockSpec((128, 256), lambda b, i: (0, 0)),
            pl.BlockSpec((DFf, 256), lambda b, i: (0, 0)),
            pl.BlockSpec((1, 256), lambda b, i: (0, 0)),
            pl.BlockSpec((256, 128), lambda b, i: (0, 0)),
            pl.BlockSpec((1, 128), lambda b, i: (0, 0)),
        ],
        out_specs=pl.BlockSpec((1, _RQ, 128), lambda b, i: (b, i, 0)),
        out_shape=jax.ShapeDtypeStruct((Bb, Nn, 128), F32),
    )(h2, fm, fg, w1at, w1bt, w1ct, b1, w2t, b2)


def _tab_body(kp_ref, p16_ref, wk_ref, bk_ref, wu_ref, tab_ref):
    kp = jnp.transpose(kp_ref[0])
    key = _dot(kp, wk_ref[...]) + bk_ref[...]
    u = _dot(p16_ref[0], wu_ref[...])
    tab_ref[0] = jnp.concatenate([key, u], axis=1)


def _tab_kern(K_prev, p16, wkt, bk, wut):
    wspec = lambda s: pl.BlockSpec(s, lambda b, i: (0, 0))
    return pl.pallas_call(
        _tab_body,
        grid=(Bb, Nn // _RQ),
        in_specs=[
            pl.BlockSpec((1, 128, _RQ), lambda b, i: (b, 0, i)),
            pl.BlockSpec((1, _RQ, 16), lambda b, i: (b, i, 0)),
            wspec((128, 64)), wspec((1, 64)), wspec((16, 64)),
        ],
        out_specs=pl.BlockSpec((1, _RQ, 128), lambda b, i: (b, i, 0)),
        out_shape=jax.ShapeDtypeStruct((Bb, Nn, 128), F32),
    )(K_prev, p16, wkt, bk, wut)


def _kqv_body(kp_ref, q_ref, wst_ref, bs_ref, w1_ref, b1_ref,
              w2_ref, b2_ref, wq_ref, bq_ref, wv_ref,
              bv_ref, val_ref, qry_ref, vt_ref):
    kp = jnp.transpose(kp_ref[0])
    q = q_ref[0]
    cat = jnp.concatenate([kp, q], axis=1)
    sc = _dot(cat, wst_ref[...]) + bs_ref[...]
    h = jnp.maximum(_dot(cat, w1_ref[...]) + b1_ref[...], 0.0)
    val = _dot(h, w2_ref[...]) + b2_ref[...] + sc
    val_ref[0] = val
    qry_ref[0] = _dot(q, wq_ref[...]) + bq_ref[...]
    vt_ref[0] = _dot(val, wv_ref[...]) + bv_ref[...]


def _kqv(kp, q, wst, bs, w1t, b1, w2t, b2, wqt, bq, wvt, bv):
    wspec = lambda s: pl.BlockSpec(s, lambda b, i: (0, 0))
    return pl.pallas_call(
        _kqv_body,
        grid=(Bb, Nn // _RQ),
        in_specs=[
            pl.BlockSpec((1, 128, _RQ), lambda b, i: (b, 0, i)),
            pl.BlockSpec((1, _RQ, 128), lambda b, i: (b, i, 0)),
            wspec((256, 128)), wspec((1, 128)),
            wspec((256, 128)), wspec((1, 128)),
            wspec((128, 128)), wspec((1, 128)),
            wspec((128, 64)), wspec((1, 64)),
            wspec((128, 64)), wspec((1, 64)),
        ],
        out_specs=[
            pl.BlockSpec((1, _RQ, 128), lambda b, i: (b, i, 0)),
            pl.BlockSpec((1, _RQ, 64), lambda b, i: (b, i, 0)),
            pl.BlockSpec((1, _RQ, 64), lambda b, i: (b, i, 0)),
        ],
        out_shape=[
            jax.ShapeDtypeStruct((Bb, Nn, 128), F32),
            jax.ShapeDtypeStruct((Bb, Nn, 64), F32),
            jax.ShapeDtypeStruct((Bb, Nn, 64), F32),
        ],
    )(kp, q, wst, bs, w1t, b1, w2t, b2, wqt, bq, wvt, bv)


_GCH = 128


def _sc_gather(table, ids):
    E = ids.shape[0]
    D = table.shape[1]
    info = plsc.get_sparse_core_info()
    nw = info.num_cores * info.num_subcores
    e_per_w = E // nw
    n_ch = e_per_w // _GCH
    mesh = plsc.VectorSubcoreMesh(core_axis_name="c", subcore_axis_name="s")

    @functools.partial(
        pl.kernel, mesh=mesh,
        out_type=jax.ShapeDtypeStruct((E, D), F32),
        scratch_types=[
            pltpu.VMEM((_GCH,), jnp.int32),
            pltpu.VMEM((_GCH, D), F32),
            pltpu.SemaphoreType.DMA,
        ],
    )
    def k(table_hbm, ids_hbm, out_hbm, idx_v, rows_v, sem):
        wid = lax.axis_index("s") * info.num_cores + lax.axis_index("c")
        base = wid * e_per_w

        def body(c, _):
            off = base + c * _GCH
            pltpu.sync_copy(ids_hbm.at[pl.ds(off, _GCH)], idx_v)
            pltpu.async_copy(table_hbm.at[idx_v], rows_v, sem).wait()
            pltpu.sync_copy(rows_v, out_hbm.at[pl.ds(off, _GCH)])
            return _

        lax.fori_loop(0, n_ch, body, 0)

    return k(table, ids)


_RA = 256


def _rep(x, c):
    return jnp.broadcast_to(x[:, None, :], (_RA, KK, c)).reshape(_RA * KK, c)


def _seg(x, c):
    return jnp.sum(x.reshape(_RA, KK, c), axis=1)


def _att_body(g_ref, q_ref, v_ref, tab_ref, idn_ref,
              bp1_ref, ps_ref, pt_ref, wp2_ref, bp2_ref,
              wa1_ref, ba1_ref, as_ref, at_ref, wa2_ref, ba2_ref,
              we_ref, be_ref, wf1_ref, bf1_ref, wf2_ref, bf2_ref,
              pw_ref, h_ref, y_ref):
    g = g_ref[0]
    qu = jnp.concatenate([q_ref[0], tab_ref[0][:, 64:128]], axis=1)
    D = _rep(qu, 128) - g
    Dp = (D + bp1_ref[...]) * ps_ref[...] + pt_ref[...]
    lane = lax.broadcasted_iota(jnp.int32, (_RA * KK, 128), 1)
    Dp = jnp.where(lane >= 64, jnp.maximum(Dp, 0.0), Dp)
    pe = _dot(Dp, wp2_ref[...]) + bp2_ref[...]
    a = _dot(Dp[:, :64] + pe, wa1_ref[...]) + ba1_ref[...]
    a = jnp.maximum(a * as_ref[...] + at_ref[...], 0.0)
    logit = _dot(a, wa2_ref[...]) + ba2_ref[...]
    e2 = jnp.exp(logit - jnp.max(logit))
    val4 = jnp.concatenate(
        [_rep(v_ref[0], 64) + pe, jnp.ones((_RA * KK, 64), F32)], axis=1)
    nd = _seg(e2 * val4, 128)
    agg = nd[:, :64] / nd[:, 64:128]
    h = _dot(agg, we_ref[...]) + be_ref[...] + idn_ref[0]
    h_ref[0] = h
    fc = jnp.maximum(_dot(h, wf1_ref[...]) + bf1_ref[...], 0.0)
    fc = _dot(fc, wf2_ref[...]) + bf2_ref[...]
    y = _dot(fc, pw_ref[...])
    y_ref[0] = y.reshape(2 * _RA, 128)


def _att(g, qt, vt, tab, idn, bp1, pbs, pbt, wp2t, bp2,
         wa1t, ba1, abs_, abt, wa2t, ba2, wet, be, wf1t, bf1, wf2t, bf2,
         pw):
    wspec = lambda s: pl.BlockSpec(s, lambda b, i: (0, 0))
    return pl.pallas_call(
        _att_body,
        grid=(Bb, Nn // _RA),
        in_specs=[
            pl.BlockSpec((1, _RA * KK, 128), lambda b, i: (b, i, 0)),
            pl.BlockSpec((1, _RA, 64), lambda b, i: (b, i, 0)),
            pl.BlockSpec((1, _RA, 64), lambda b, i: (b, i, 0)),
            pl.BlockSpec((1, _RA, 128), lambda b, i: (b, i, 0)),
            pl.BlockSpec((1, _RA, 128), lambda b, i: (b, i, 0)),
            wspec((1, 128)), wspec((1, 128)), wspec((1, 128)),
            wspec((128, 64)), wspec((1, 64)),
            wspec((64, 256)), wspec((1, 256)), wspec((1, 256)), wspec((1, 256)),
            wspec((256, 128)), wspec((1, 128)),
            wspec((64, 128)), wspec((1, 128)),
            wspec((128, 64)), wspec((1, 64)),
            wspec((64, 32)), wspec((1, 32)),
            wspec((32, 256)),
        ],
        out_specs=[
            pl.BlockSpec((1, _RA, 128), lambda b, i: (b, i, 0)),
            pl.BlockSpec((1, 2 * _RA, 128), lambda b, i: (b, i, 0)),
        ],
        out_shape=[
            jax.ShapeDtypeStruct((Bb, Nn, 128), F32),
            jax.ShapeDtypeStruct((Bb, 2 * Nn, 128), F32),
        ],
    )(g, qt, vt, tab, idn, bp1, pbs, pbt, wp2t, bp2,
      wa1t, ba1, abs_, abt, wa2t, ba2, wet, be, wf1t, bf1, wf2t, bf2, pw)


_RF = 512


def _fin_body(fc_ref, h_ref, pts_ref, wsa_ref, wsb_ref, bs_ref,
              w1a_ref, w1b_ref, b1_ref, w2_ref, b2_ref,
              wd1_ref, bd1_ref, wd2_ref, bd2_ref, kc_ref, po_ref):
    fcx = fc_ref[0]
    hh = jnp.broadcast_to(h_ref[0][:, None, :], (_RF // 2, 2, 128)
                          ).reshape(_RF, 128)
    sc = _dot(fcx, wsa_ref[...]) + _dot(hh, wsb_ref[...]) + bs_ref[...]
    h = jnp.maximum(_dot(fcx, w1a_ref[...]) + _dot(hh, w1b_ref[...])
                    + b1_ref[...], 0.0)
    kc = _dot(h, w2_ref[...]) + b2_ref[...] + sc
    kc_ref[0] = jnp.transpose(kc)
    r = jnp.maximum(kc, 0.0)
    d1 = jnp.maximum(_dot(r, wd1_ref[...]) + bd1_ref[...], 0.0)
    dl = _dot(d1, wd2_ref[...]) + bd2_ref[...]
    pp = jnp.broadcast_to(pts_ref[0][:, None, :], (_RF // 2, 2, 8)
                          ).reshape(_RF, 8)
    po_ref[0] = jnp.transpose(pp + jnp.tanh(dl))


def _fin(fc_rows, H, pts8, wsat, wsbt, bs, w1at, w1bt, b1, w2t, b2,
         wd1t, bd1, wd2t, bd2):
    wspec = lambda s: pl.BlockSpec(s, lambda b, i: (0, 0))
    N2 = 2 * Nn
    return pl.pallas_call(
        _fin_body,
        grid=(Bb, N2 // _RF),
        in_specs=[
            pl.BlockSpec((1, _RF, 128), lambda b, i: (b, i, 0)),
            pl.BlockSpec((1, _RF // 2, 128), lambda b, i: (b, i, 0)),
            pl.BlockSpec((1, _RF // 2, 8), lambda b, i: (b, i, 0)),
            wspec((128, 128)), wspec((128, 128)), wspec((1, 128)),
            wspec((128, 128)), wspec((128, 128)), wspec((1, 128)),
            wspec((128, 128)), wspec((1, 128)),
            wspec((128, 64)), wspec((1, 64)),
            wspec((64, 8)), wspec((1, 8)),
        ],
        out_specs=[
            pl.BlockSpec((1, 128, _RF), lambda b, i: (b, 0, i)),
            pl.BlockSpec((1, 8, _RF), lambda b, i: (b, 0, i)),
        ],
        out_shape=[
            jax.ShapeDtypeStruct((Bb, 128, N2), F32),
            jax.ShapeDtypeStruct((Bb, 8, N2), F32),
        ],
    )(fc_rows, H, pts8, wsat, wsbt, bs, w1at, w1bt, b1, w2t, b2,
      wd1t, bd1, wd2t, bd2)


def _bn_fold(p, pre):
    s = p[pre + '_g'] / jnp.sqrt(p[pre + '_v'] + EPS)
    t = p[pre + '_b'] - p[pre + '_m'] * s
    return s[None, :], t[None, :]


def _bn_fold2(p, pre):
    s = p[pre + 'g'] / jnp.sqrt(p[pre + 'v'] + EPS)
    t = p[pre + 'b'] - p[pre + 'm'] * s
    return s[None, :], t[None, :]


def _row(v):
    return v[None, :]


def kernel(pcd_prev, feat_global, K_prev, params):
    p = params
    pts = jnp.transpose(pcd_prev, (0, 2, 1))
    pts8 = jnp.pad(pts, ((0, 0), (0, 0), (0, 5)))
    pts16 = jnp.pad(pts, ((0, 0), (0, 0), (0, 13)))

    idx, idx2 = _knn(pts8)

    tab = _tab_kern(K_prev, pts16, p['st_Wk'].T, _row(p['st_bk']),
                    jnp.pad(p['st_pos_W1'].T, ((0, 13), (0, 0))))
    flat_ids = (idx2 + (jnp.arange(Bb, dtype=jnp.int32) * Nn)[:, None, None]
                ).reshape(-1)
    g = _sc_gather(tab.reshape(Bb * Nn, 128), flat_ids
                   ).reshape(Bb, Nn * KK, 128)

    x = _mlp1(pts8.reshape(Bb * Nn, 8),
              jnp.pad(p['mlp1_W1'].T, ((0, 5), (0, 0))), _row(p['mlp1_b1']),
              p['mlp1_W2'].T, _row(p['mlp1_b2'])).reshape(Bb, Nn, 128)

    s1, t1 = _bn_fold(p, 'bn1')
    h1, A = _sage1(x, idx, p['sage1_Wl'].T, _row(p['sage1_bl']),
                   p['sage1_Wr'].T, s1, t1)
    s2, t2 = _bn_fold(p, 'bn2')
    h2 = _sage2(h1, A, p['sage2_Wl'].T, _row(p['sage2_bl']),
                p['sage2_Wr'].T, s2, t2)

    fm = _fmax(h2)
    fg = jnp.transpose(feat_global, (0, 2, 1))
    W1 = p['mlp2_W1']
    Q = _qkern(h2, fm, fg,
               W1[:, :128].T, W1[:, 128:256].T, W1[:, 256:].T,
               _row(p['mlp2_b1']), p['mlp2_W2'].T, _row(p['mlp2_b2']))

    value, query_t, value_t = _kqv(
        K_prev, Q,
        p['st_mlpv_Ws'].T, _row(p['st_mlpv_bs']),
        p['st_mlpv_W1'].T, _row(p['st_mlpv_b1']),
        p['st_mlpv_W2'].T, _row(p['st_mlpv_b2']),
        p['st_Wq'].T, _row(p['st_bq']),
        p['st_Wv'].T, _row(p['st_bv']))

    pbs, pbt = _bn_fold2(p, 'st_pos_bn')
    abs_, abt = _bn_fold2(p, 'st_att_bn')
    z64 = jnp.zeros((1, 64), F32)
    o64 = jnp.ones((1, 64), F32)
    wa2d = p['st_att_W2'].T
    H, fc_rows = _att(
        g, query_t, value_t, tab, value,
        jnp.concatenate([z64, _row(p['st_pos_b1'])], axis=1),
        jnp.concatenate([o64, pbs], axis=1),
        jnp.concatenate([z64, pbt], axis=1),
        jnp.concatenate([jnp.zeros((64, 64), F32), p['st_pos_W2'].T], axis=0),
        _row(p['st_pos_b2']),
        p['st_att_W1'].T, _row(p['st_att_b1']), abs_, abt,
        jnp.concatenate([wa2d, wa2d], axis=1),
        jnp.concatenate([_row(p['st_att_b2']), _row(p['st_att_b2'])], axis=1),
        p['st_We'].T, _row(p['st_be']),
        p['mlpps_W1'].T, _row(p['mlpps_b1']),
        p['mlpps_W2'].T, _row(p['mlpps_b2']),
        jnp.transpose(p['ps_W'], (0, 2, 1)).reshape(32, 256))

    Wdfs = p['mlpdf_Ws'].T
    Wdf1 = p['mlpdf_W1'].T
    kc, po = _fin(fc_rows, H, pts8,
                  Wdfs[:128], Wdfs[128:], _row(p['mlpdf_bs']),
                  Wdf1[:128], Wdf1[128:], _row(p['mlpdf_b1']),
                  p['mlpdf_W2'].T, _row(p['mlpdf_b2']),
                  p['mlpd_W1'].T, _row(p['mlpd_b1']),
                  jnp.pad(p['mlpd_W2'].T, ((0, 0), (0, 5))),
                  jnp.pad(_row(p['mlpd_b2']), ((0, 0), (0, 5))))

    return po[:, :3, :], kc


# speedup vs baseline: 1.6737x; 1.0924x over previous
import functools

import jax
import jax.numpy as jnp
from jax import lax
from jax.experimental import pallas as pl
from jax.experimental.pallas import tpu as pltpu
from jax.experimental.pallas import tpu_sc as plsc

Bb, Nn, DFf, KK = 4, 2048, 512, 16
EPS = 1e-5
F32 = jnp.float32


def _hi(x):
    return x


def _dotT(a, b):
    return lax.dot_general(a, b, (((1,), (1,)), ((), ())),
                           preferred_element_type=F32)


def _dot(a, b):
    return lax.dot_general(a, b, (((1,), (0,)), ((), ())),
                           preferred_element_type=F32)


_RBK = 256


def _knn_body(ptsb_ref, ptsf_ref, ones_ref, a_ref, idx2_ref):
    pb = ptsb_ref[0]
    pf = ptsf_ref[0]
    sq_b = jnp.sum(pb * pb, axis=1, keepdims=True)
    sq_f = _dotT(ones_ref[...], pf * pf)
    g = _dotT(pb, pf)
    d = sq_b + sq_f - 2.0 * g
    col = lax.broadcasted_iota(jnp.int32, (_RBK, Nn), 1)
    big = jnp.float32(1e30)
    i = pl.program_id(1)
    n_self = i * _RBK + lax.broadcasted_iota(jnp.int32, (_RBK, 1), 0)
    dd = jnp.where(col == n_self, big, d)
    colf = col.astype(F32)
    ams = []
    for _ in range(KK):
        dmin = jnp.min(dd, axis=1, keepdims=True)
        eqm = dd == dmin
        am = jnp.min(jnp.where(eqm, colf, jnp.float32(Nn)), axis=1,
                     keepdims=True)
        ams.append(am)
        dd = jnp.where(eqm, big, dd)
    v = jnp.concatenate(ams, axis=1).astype(jnp.int32)
    a_ref[0] = jnp.where(col == n_self, 0.0,
                         jnp.where(dd == big, 1.0, 0.0)).astype(jnp.bfloat16)
    idx2_ref[0] = jnp.concatenate([n_self, v[:, :KK - 1]], axis=1)


def _knn(pts8):
    grid = (Bb, Nn // _RBK)
    out = pl.pallas_call(
        _knn_body,
        grid=grid,
        in_specs=[
            pl.BlockSpec((1, _RBK, 8), lambda b, i: (b, i, 0)),
            pl.BlockSpec((1, Nn, 8), lambda b, i: (b, 0, 0)),
            pl.BlockSpec((1, 8), lambda b, i: (0, 0)),
        ],
        out_specs=[
            pl.BlockSpec((1, _RBK, Nn), lambda b, i: (b, i, 0)),
            pl.BlockSpec((1, _RBK, KK), lambda b, i: (b, i, 0)),
        ],
        out_shape=[
            jax.ShapeDtypeStruct((Bb, Nn, Nn), jnp.bfloat16),
            jax.ShapeDtypeStruct((Bb, Nn, KK), jnp.int32),
        ],
    )(pts8, pts8, jnp.ones((1, 8), F32))
    return out


def _mlp1_body(x_ref, w1_ref, b1_ref, w2_ref, b2_ref, o_ref):
    h = jnp.maximum(_dot(x_ref[...], w1_ref[...]) + b1_ref[...], 0.0)
    o_ref[...] = _dot(h, w2_ref[...]) + b2_ref[...]


def _mlp1(xrows, w1t, b1, w2t, b2):
    R = xrows.shape[0]
    RB = 1024
    return pl.pallas_call(
        _mlp1_body,
        grid=(R // RB,),
        in_specs=[
            pl.BlockSpec((RB, 8), lambda i: (i, 0)),
            pl.BlockSpec(w1t.shape, lambda i: (0, 0)),
            pl.BlockSpec(b1.shape, lambda i: (0, 0)),
            pl.BlockSpec(w2t.shape, lambda i: (0, 0)),
            pl.BlockSpec(b2.shape, lambda i: (0, 0)),
        ],
        out_specs=pl.BlockSpec((RB, 128), lambda i: (i, 0)),
        out_shape=jax.ShapeDtypeStruct((R, 128), F32),
    )(xrows, w1t, b1, w2t, b2)


_RBS = 256


def _sage_body(a_ref, xf_ref, xb_ref, wl_ref, wr_ref, cb_ref, sc_ref,
               sh_ref, o_ref, *, residual):
    acc = a_ref[0].astype(F32)
    mean = _dot(acc, xf_ref[0]) * (1.0 / KK)
    xb = xb_ref[0]
    y = _dot(mean, wl_ref[...]) + cb_ref[...] + _dot(xb, wr_ref[...])
    y = jnp.maximum(y * sc_ref[...] + sh_ref[...], 0.0)
    if residual:
        y = y + xb
    o_ref[0] = y


def _sage(x, A, wlt, bl, wrt, bn_s, bn_t, residual):
    return pl.pallas_call(
        functools.partial(_sage_body, residual=residual),
        grid=(Bb, Nn // _RBS),
        in_specs=[
            pl.BlockSpec((1, _RBS, Nn), lambda b, i: (b, i, 0)),
            pl.BlockSpec((1, Nn, 128), lambda b, i: (b, 0, 0)),
            pl.BlockSpec((1, _RBS, 128), lambda b, i: (b, i, 0)),
            pl.BlockSpec((128, 128), lambda b, i: (0, 0)),
            pl.BlockSpec((128, 128), lambda b, i: (0, 0)),
            pl.BlockSpec((1, 128), lambda b, i: (0, 0)),
            pl.BlockSpec((1, 128), lambda b, i: (0, 0)),
            pl.BlockSpec((1, 128), lambda b, i: (0, 0)),
        ],
        out_specs=pl.BlockSpec((1, _RBS, 128), lambda b, i: (b, i, 0)),
        out_shape=jax.ShapeDtypeStruct((Bb, Nn, 128), F32),
    )(A, x, x, wlt, wrt, bl, bn_s, bn_t)


def _fmax_body(x_ref, o_ref):
    o_ref[0] = jnp.max(x_ref[0], axis=0, keepdims=True)


def _fmax(h2):
    return pl.pallas_call(
        _fmax_body,
        grid=(Bb,),
        in_specs=[pl.BlockSpec((1, Nn, 128), lambda b: (b, 0, 0))],
        out_specs=pl.BlockSpec((1, 1, 128), lambda b: (b, 0, 0)),
        out_shape=jax.ShapeDtypeStruct((Bb, 1, 128), F32),
    )(h2)


_RQ = 512


def _q_body(x_ref, fm_ref, fg_ref, w1a_ref, w1b_ref, w1c_ref, b1_ref,
            w2_ref, b2_ref, o_ref):
    c = (_dot(fm_ref[0], w1b_ref[...]) + _dot(fg_ref[0], w1c_ref[...])
         + b1_ref[...])
    h = jnp.maximum(_dot(x_ref[0], w1a_ref[...]) + c, 0.0)
    o_ref[0] = _dot(h, w2_ref[...]) + b2_ref[...]


def _qkern(h2, fm, fg, w1at, w1bt, w1ct, b1, w2t, b2):
    return pl.pallas_call(
        _q_body,
        grid=(Bb, Nn // _RQ),
        in_specs=[
            pl.BlockSpec((1, _RQ, 128), lambda b, i: (b, i, 0)),
            pl.BlockSpec((1, 1, 128), lambda b, i: (b, 0, 0)),
            pl.BlockSpec((1, 1, DFf), lambda b, i: (b, 0, 0)),
            pl.BlockSpec((128, 256), lambda b, i: (0, 0)),
            pl.BlockSpec((128, 256), lambda b, i: (0, 0)),
            pl.BlockSpec((DFf, 256), lambda b, i: (0, 0)),
            pl.BlockSpec((1, 256), lambda b, i: (0, 0)),
            pl.BlockSpec((256, 128), lambda b, i: (0, 0)),
            pl.BlockSpec((1, 128), lambda b, i: (0, 0)),
        ],
        out_specs=pl.BlockSpec((1, _RQ, 128), lambda b, i: (b, i, 0)),
        out_shape=jax.ShapeDtypeStruct((Bb, Nn, 128), F32),
    )(h2, fm, fg, w1at, w1bt, w1ct, b1, w2t, b2)


def _tab_body(kp_ref, p16_ref, wk_ref, bk_ref, wu_ref, tab_ref):
    kp = jnp.transpose(kp_ref[0])
    key = _dot(kp, wk_ref[...]) + bk_ref[...]
    u = _dot(p16_ref[0], wu_ref[...])
    tab_ref[0] = jnp.concatenate([key, u], axis=1)


def _tab_kern(K_prev, p16, wkt, bk, wut):
    wspec = lambda s: pl.BlockSpec(s, lambda b, i: (0, 0))
    return pl.pallas_call(
        _tab_body,
        grid=(Bb, Nn // _RQ),
        in_specs=[
            pl.BlockSpec((1, 128, _RQ), lambda b, i: (b, 0, i)),
            pl.BlockSpec((1, _RQ, 16), lambda b, i: (b, i, 0)),
            wspec((128, 64)), wspec((1, 64)), wspec((16, 64)),
        ],
        out_specs=pl.BlockSpec((1, _RQ, 128), lambda b, i: (b, i, 0)),
        out_shape=jax.ShapeDtypeStruct((Bb, Nn, 128), F32),
    )(K_prev, p16, wkt, bk, wut)


def _kqv_body(kp_ref, q_ref, wst_ref, bs_ref, w1_ref, b1_ref,
              w2_ref, b2_ref, wq_ref, bq_ref, wv_ref,
              bv_ref, val_ref, qry_ref, vt_ref):
    kp = jnp.transpose(kp_ref[0])
    q = q_ref[0]
    cat = jnp.concatenate([kp, q], axis=1)
    sc = _dot(cat, wst_ref[...]) + bs_ref[...]
    h = jnp.maximum(_dot(cat, w1_ref[...]) + b1_ref[...], 0.0)
    val = _dot(h, w2_ref[...]) + b2_ref[...] + sc
    val_ref[0] = val
    qry_ref[0] = _dot(q, wq_ref[...]) + bq_ref[...]
    vt_ref[0] = _dot(val, wv_ref[...]) + bv_ref[...]


def _kqv(kp, q, wst, bs, w1t, b1, w2t, b2, wqt, bq, wvt, bv):
    wspec = lambda s: pl.BlockSpec(s, lambda b, i: (0, 0))
    return pl.pallas_call(
        _kqv_body,
        grid=(Bb, Nn // _RQ),
        in_specs=[
            pl.BlockSpec((1, 128, _RQ), lambda b, i: (b, 0, i)),
            pl.BlockSpec((1, _RQ, 128), lambda b, i: (b, i, 0)),
            wspec((256, 128)), wspec((1, 128)),
            wspec((256, 128)), wspec((1, 128)),
            wspec((128, 128)), wspec((1, 128)),
            wspec((128, 64)), wspec((1, 64)),
            wspec((128, 64)), wspec((1, 64)),
        ],
        out_specs=[
            pl.BlockSpec((1, _RQ, 128), lambda b, i: (b, i, 0)),
            pl.BlockSpec((1, _RQ, 64), lambda b, i: (b, i, 0)),
            pl.BlockSpec((1, _RQ, 64), lambda b, i: (b, i, 0)),
        ],
        out_shape=[
            jax.ShapeDtypeStruct((Bb, Nn, 128), F32),
            jax.ShapeDtypeStruct((Bb, Nn, 64), F32),
            jax.ShapeDtypeStruct((Bb, Nn, 64), F32),
        ],
    )(kp, q, wst, bs, w1t, b1, w2t, b2, wqt, bq, wvt, bv)


_GCH = 128


def _sc_gather(table, ids):
    E = ids.shape[0]
    D = table.shape[1]
    info = plsc.get_sparse_core_info()
    nw = info.num_cores * info.num_subcores
    e_per_w = E // nw
    n_ch = e_per_w // _GCH
    mesh = plsc.VectorSubcoreMesh(core_axis_name="c", subcore_axis_name="s")

    @functools.partial(
        pl.kernel, mesh=mesh,
        out_type=jax.ShapeDtypeStruct((E, D), F32),
        scratch_types=[
            pltpu.VMEM((_GCH,), jnp.int32),
            pltpu.VMEM((_GCH, D), F32),
            pltpu.SemaphoreType.DMA,
        ],
    )
    def k(table_hbm, ids_hbm, out_hbm, idx_v, rows_v, sem):
        wid = lax.axis_index("s") * info.num_cores + lax.axis_index("c")
        base = wid * e_per_w

        def body(c, _):
            off = base + c * _GCH
            pltpu.sync_copy(ids_hbm.at[pl.ds(off, _GCH)], idx_v)
            pltpu.async_copy(table_hbm.at[idx_v], rows_v, sem).wait()
            pltpu.sync_copy(rows_v, out_hbm.at[pl.ds(off, _GCH)])
            return _

        lax.fori_loop(0, n_ch, body, 0)

    return k(table, ids)


_RA = 256


def _rep(x, c):
    return jnp.broadcast_to(x[:, None, :], (_RA, KK, c)).reshape(_RA * KK, c)


def _seg(x, c):
    return jnp.sum(x.reshape(_RA, KK, c), axis=1)


def _att_body(g_ref, q_ref, v_ref, tab_ref, idn_ref,
              bp1_ref, ps_ref, pt_ref, wp2_ref, bp2_ref,
              wa1_ref, ba1_ref, as_ref, at_ref, wa2_ref, ba2_ref,
              we_ref, be_ref, wf1_ref, bf1_ref, wf2_ref, bf2_ref,
              pw_ref, h_ref, y_ref):
    g = g_ref[0]
    qu = jnp.concatenate([q_ref[0], tab_ref[0][:, 64:128]], axis=1)
    D = _rep(qu, 128) - g
    Dp = (D + bp1_ref[...]) * ps_ref[...] + pt_ref[...]
    lane = lax.broadcasted_iota(jnp.int32, (_RA * KK, 128), 1)
    Dp = jnp.where(lane >= 64, jnp.maximum(Dp, 0.0), Dp)
    pe = _dot(Dp, wp2_ref[...]) + bp2_ref[...]
    a = _dot(Dp[:, :64] + pe, wa1_ref[...]) + ba1_ref[...]
    a = jnp.maximum(a * as_ref[...] + at_ref[...], 0.0)
    logit = _dot(a, wa2_ref[...]) + ba2_ref[...]
    e2 = jnp.exp(logit - jnp.max(logit))
    val4 = jnp.concatenate(
        [_rep(v_ref[0], 64) + pe, jnp.ones((_RA * KK, 64), F32)], axis=1)
    nd = _seg(e2 * val4, 128)
    agg = nd[:, :64] / nd[:, 64:128]
    h = _dot(agg, we_ref[...]) + be_ref[...] + idn_ref[0]
    h_ref[0] = h
    fc = jnp.maximum(_dot(h, wf1_ref[...]) + bf1_ref[...], 0.0)
    fc = _dot(fc, wf2_ref[...]) + bf2_ref[...]
    y = _dot(fc, pw_ref[...])
    y_ref[0] = y.reshape(2 * _RA, 128)


def _att(g, qt, vt, tab, idn, bp1, pbs, pbt, wp2t, bp2,
         wa1t, ba1, abs_, abt, wa2t, ba2, wet, be, wf1t, bf1, wf2t, bf2,
         pw):
    wspec = lambda s: pl.BlockSpec(s, lambda b, i: (0, 0))
    return pl.pallas_call(
        _att_body,
        grid=(Bb, Nn // _RA),
        in_specs=[
            pl.BlockSpec((1, _RA * KK, 128), lambda b, i: (b, i, 0)),
            pl.BlockSpec((1, _RA, 64), lambda b, i: (b, i, 0)),
            pl.BlockSpec((1, _RA, 64), lambda b, i: (b, i, 0)),
            pl.BlockSpec((1, _RA, 128), lambda b, i: (b, i, 0)),
            pl.BlockSpec((1, _RA, 128), lambda b, i: (b, i, 0)),
            wspec((1, 128)), wspec((1, 128)), wspec((1, 128)),
            wspec((128, 64)), wspec((1, 64)),
            wspec((64, 256)), wspec((1, 256)), wspec((1, 256)), wspec((1, 256)),
            wspec((256, 128)), wspec((1, 128)),
            wspec((64, 128)), wspec((1, 128)),
            wspec((128, 64)), wspec((1, 64)),
            wspec((64, 32)), wspec((1, 32)),
            wspec((32, 256)),
        ],
        out_specs=[
            pl.BlockSpec((1, _RA, 128), lambda b, i: (b, i, 0)),
            pl.BlockSpec((1, 2 * _RA, 128), lambda b, i: (b, i, 0)),
        ],
        out_shape=[
            jax.ShapeDtypeStruct((Bb, Nn, 128), F32),
            jax.ShapeDtypeStruct((Bb, 2 * Nn, 128), F32),
        ],
    )(g, qt, vt, tab, idn, bp1, pbs, pbt, wp2t, bp2,
      wa1t, ba1, abs_, abt, wa2t, ba2, wet, be, wf1t, bf1, wf2t, bf2, pw)


_RF = 512


def _fin_body(fc_ref, h_ref, pts_ref, wsa_ref, wsb_ref, bs_ref,
              w1a_ref, w1b_ref, b1_ref, w2_ref, b2_ref,
              wd1_ref, bd1_ref, wd2_ref, bd2_ref, kc_ref, po_ref):
    fcx = fc_ref[0]
    hh = jnp.broadcast_to(h_ref[0][:, None, :], (_RF // 2, 2, 128)
                          ).reshape(_RF, 128)
    sc = _dot(fcx, wsa_ref[...]) + _dot(hh, wsb_ref[...]) + bs_ref[...]
    h = jnp.maximum(_dot(fcx, w1a_ref[...]) + _dot(hh, w1b_ref[...])
                    + b1_ref[...], 0.0)
    kc = _dot(h, w2_ref[...]) + b2_ref[...] + sc
    kc_ref[0] = jnp.transpose(kc)
    r = jnp.maximum(kc, 0.0)
    d1 = jnp.maximum(_dot(r, wd1_ref[...]) + bd1_ref[...], 0.0)
    dl = _dot(d1, wd2_ref[...]) + bd2_ref[...]
    pp = jnp.broadcast_to(pts_ref[0][:, None, :], (_RF // 2, 2, 8)
                          ).reshape(_RF, 8)
    po_ref[0] = jnp.transpose(pp + jnp.tanh(dl))


def _fin(fc_rows, H, pts8, wsat, wsbt, bs, w1at, w1bt, b1, w2t, b2,
         wd1t, bd1, wd2t, bd2):
    wspec = lambda s: pl.BlockSpec(s, lambda b, i: (0, 0))
    N2 = 2 * Nn
    return pl.pallas_call(
        _fin_body,
        grid=(Bb, N2 // _RF),
        in_specs=[
            pl.BlockSpec((1, _RF, 128), lambda b, i: (b, i, 0)),
            pl.BlockSpec((1, _RF // 2, 128), lambda b, i: (b, i, 0)),
            pl.BlockSpec((1, _RF // 2, 8), lambda b, i: (b, i, 0)),
            wspec((128, 128)), wspec((128, 128)), wspec((1, 128)),
            wspec((128, 128)), wspec((128, 128)), wspec((1, 128)),
            wspec((128, 128)), wspec((1, 128)),
            wspec((128, 64)), wspec((1, 64)),
            wspec((64, 8)), wspec((1, 8)),
        ],
        out_specs=[
            pl.BlockSpec((1, 128, _RF), lambda b, i: (b, 0, i)),
            pl.BlockSpec((1, 8, _RF), lambda b, i: (b, 0, i)),
        ],
        out_shape=[
            jax.ShapeDtypeStruct((Bb, 128, N2), F32),
            jax.ShapeDtypeStruct((Bb, 8, N2), F32),
        ],
    )(fc_rows, H, pts8, wsat, wsbt, bs, w1at, w1bt, b1, w2t, b2,
      wd1t, bd1, wd2t, bd2)


def _bn_fold(p, pre):
    s = p[pre + '_g'] / jnp.sqrt(p[pre + '_v'] + EPS)
    t = p[pre + '_b'] - p[pre + '_m'] * s
    return s[None, :], t[None, :]


def _bn_fold2(p, pre):
    s = p[pre + 'g'] / jnp.sqrt(p[pre + 'v'] + EPS)
    t = p[pre + 'b'] - p[pre + 'm'] * s
    return s[None, :], t[None, :]


def _row(v):
    return v[None, :]


def kernel(pcd_prev, feat_global, K_prev, params):
    p = params
    pts = jnp.transpose(pcd_prev, (0, 2, 1))
    pts8 = jnp.pad(pts, ((0, 0), (0, 0), (0, 5)))
    pts16 = jnp.pad(pts, ((0, 0), (0, 0), (0, 13)))

    A, idx2 = _knn(pts8)

    tab = _tab_kern(K_prev, pts16, p['st_Wk'].T, _row(p['st_bk']),
                    jnp.pad(p['st_pos_W1'].T, ((0, 13), (0, 0))))
    flat_ids = (idx2 + (jnp.arange(Bb, dtype=jnp.int32) * Nn)[:, None, None]
                ).reshape(-1)
    g = _sc_gather(tab.reshape(Bb * Nn, 128), flat_ids
                   ).reshape(Bb, Nn * KK, 128)

    x = _mlp1(pts8.reshape(Bb * Nn, 8),
              jnp.pad(p['mlp1_W1'].T, ((0, 5), (0, 0))), _row(p['mlp1_b1']),
              p['mlp1_W2'].T, _row(p['mlp1_b2'])).reshape(Bb, Nn, 128)

    s1, t1 = _bn_fold(p, 'bn1')
    h1 = _sage(x, A, p['sage1_Wl'].T, _row(p['sage1_bl']),
               p['sage1_Wr'].T, s1, t1, residual=False)
    s2, t2 = _bn_fold(p, 'bn2')
    h2 = _sage(h1, A, p['sage2_Wl'].T, _row(p['sage2_bl']),
               p['sage2_Wr'].T, s2, t2, residual=True)

    fm = _fmax(h2)
    fg = jnp.transpose(feat_global, (0, 2, 1))
    W1 = p['mlp2_W1']
    Q = _qkern(h2, fm, fg,
               W1[:, :128].T, W1[:, 128:256].T, W1[:, 256:].T,
               _row(p['mlp2_b1']), p['mlp2_W2'].T, _row(p['mlp2_b2']))

    value, query_t, value_t = _kqv(
        K_prev, Q,
        p['st_mlpv_Ws'].T, _row(p['st_mlpv_bs']),
        p['st_mlpv_W1'].T, _row(p['st_mlpv_b1']),
        p['st_mlpv_W2'].T, _row(p['st_mlpv_b2']),
        p['st_Wq'].T, _row(p['st_bq']),
        p['st_Wv'].T, _row(p['st_bv']))

    pbs, pbt = _bn_fold2(p, 'st_pos_bn')
    abs_, abt = _bn_fold2(p, 'st_att_bn')
    z64 = jnp.zeros((1, 64), F32)
    o64 = jnp.ones((1, 64), F32)
    wa2d = p['st_att_W2'].T
    H, fc_rows = _att(
        g, query_t, value_t, tab, value,
        jnp.concatenate([z64, _row(p['st_pos_b1'])], axis=1),
        jnp.concatenate([o64, pbs], axis=1),
        jnp.concatenate([z64, pbt], axis=1),
        jnp.concatenate([jnp.zeros((64, 64), F32), p['st_pos_W2'].T], axis=0),
        _row(p['st_pos_b2']),
        p['st_att_W1'].T, _row(p['st_att_b1']), abs_, abt,
        jnp.concatenate([wa2d, wa2d], axis=1),
        jnp.concatenate([_row(p['st_att_b2']), _row(p['st_att_b2'])], axis=1),
        p['st_We'].T, _row(p['st_be']),
        p['mlpps_W1'].T, _row(p['mlpps_b1']),
        p['mlpps_W2'].T, _row(p['mlpps_b2']),
        jnp.transpose(p['ps_W'], (0, 2, 1)).reshape(32, 256))

    Wdfs = p['mlpdf_Ws'].T
    Wdf1 = p['mlpdf_W1'].T
    kc, po = _fin(fc_rows, H, pts8,
                  Wdfs[:128], Wdfs[128:], _row(p['mlpdf_bs']),
                  Wdf1[:128], Wdf1[128:], _row(p['mlpdf_b1']),
                  p['mlpdf_W2'].T, _row(p['mlpdf_b2']),
                  p['mlpd_W1'].T, _row(p['mlpd_b1']),
                  jnp.pad(p['mlpd_W2'].T, ((0, 0), (0, 5))),
                  jnp.pad(_row(p['mlpd_b2']), ((0, 0), (0, 5))))

    return po[:, :3, :], kc
